# baseline (speedup 1.0000x reference)
import functools

import jax
import jax.numpy as jnp
from jax import lax
from jax.experimental import pallas as pl
from jax.experimental.pallas import tpu as pltpu
from jax.experimental.pallas import tpu_sc as plsc

NC = 2
NS = 16
NW = NC * NS
L = 16

_N = 128 * 32768
_PER_W = _N // NW
_CHUNK = 16384
_NCHUNKS = _PER_W // _CHUNK
_KK = 64 * 128


def _wid():
    return lax.axis_index("s") * NC + lax.axis_index("c")


def _lanes():
    return lax.iota(jnp.int32, L)


def _ku16(v):
    i = lax.bitcast_convert_type(v, jnp.int32)
    k = i ^ ((i >> 31) & jnp.int32(0x7FFFFFFF))
    return lax.bitcast_convert_type(k, jnp.uint32) ^ jnp.uint32(0x80000000)


def _zero(ref, nwords):
    z = jnp.zeros((L,), jnp.int32)

    def b(i, _):
        ref[pl.ds(i * L, L)] = z
        return 0

    lax.fori_loop(0, nwords // L, b, 0, unroll=4)


def _scal(v):
    return jnp.max(v)


def _at(v, lane):
    return jnp.sum(jnp.where(_lanes() == lane, v, jnp.zeros_like(v)))


def _pick(v, kkt, running):
    r = lax.rev(v, (0,))
    cs = plsc.cumsum(r)
    m = (running + cs) >= kkt
    lb = _scal(plsc.all_reduce_ffs(m))
    above = running + _at(cs, lb) - _at(r, lb)
    return jnp.int32(15) - lb, above


def _desc_select(mg, gsum, nbuckets, kkt):
    li = _lanes()
    if nbuckets == 4096:
        def bg(g, _):
            acc = jnp.zeros((L,), jnp.int32)
            for l in range(L):
                acc = acc + plsc.load_gather(mg, [(g * L + li) * L + l])
            gsum[pl.ds(g * L, L)] = acc
            return 0

        lax.fori_loop(0, 16, bg, 0)
        ss = jnp.zeros((L,), jnp.int32)
        for l in range(L):
            ss = ss + plsc.load_gather(gsum, [li * L + l])
        s_star, ab0 = _pick(ss, kkt, jnp.int32(0))
        gvec = gsum[pl.ds(s_star * L, L)]
        g_in, ab1 = _pick(gvec, kkt, ab0)
        g_star = s_star * L + g_in
        bvec = mg[pl.ds(g_star * L, L)]
        b_in, ab2 = _pick(bvec, kkt, ab1)
        return g_star * L + b_in, ab2
    else:
        ss = jnp.zeros((L,), jnp.int32)
        for l in range(L):
            ss = ss + plsc.load_gather(mg, [li * L + l])
        g_star, ab0 = _pick(ss, kkt, jnp.int32(0))
        bvec = mg[pl.ds(g_star * L, L)]
        b_in, ab1 = _pick(bvec, kkt, ab0)
        return g_star * L + b_in, ab1


def _global_merge(h_hbm, tmp, mg):
    _zero(mg, 4096)
    for cc in range(8):
        pltpu.sync_copy(h_hbm.at[pl.ds(cc * 16384, 16384)], tmp)

        def b(g, _):
            acc = mg[pl.ds(g * L, L)]
            for w in range(4):
                acc = acc + tmp[pl.ds(w * 4096 + g * L, L)]
            mg[pl.ds(g * L, L)] = acc
            return 0

        lax.fori_loop(0, 256, b, 0)


def _lane_merge(subhist, out_ref, nbuckets):
    def b(g, _):
        acc = jnp.zeros((L,), jnp.int32)
        for l in range(L):
            acc = acc + subhist[pl.ds(l * nbuckets + g * L, L)]
        out_ref[pl.ds(g * L, L)] = acc
        return 0

    lax.fori_loop(0, nbuckets // L, b, 0)


def _hist_stream(x_hbm, buf, subhist, wid, bucket_and_mask):
    ones = jnp.ones((L,), jnp.int32)
    li = _lanes()

    for c in range(_NCHUNKS):
        base = wid * _PER_W + c * _CHUNK
        pltpu.sync_copy(x_hbm.at[pl.ds(base, _CHUNK)], buf)

        def inner(j, _):
            v = buf[pl.ds(j * L, L)]
            ku = _ku16(v)
            bkt, msk, nb = bucket_and_mask(ku)
            idx = li * jnp.int32(nb) + bkt
            if msk is None:
                plsc.addupdate_scatter(subhist, [idx], ones)
            else:
                plsc.addupdate_scatter(subhist, [idx], ones, mask=msk)
            return 0

        lax.fori_loop(0, _CHUNK // L, inner, 0, unroll=8)


@functools.cache
def _build_passes():
    mesh = plsc.VectorSubcoreMesh(core_axis_name="c", subcore_axis_name="s")

    @functools.partial(
        pl.kernel,
        out_type=jax.ShapeDtypeStruct((NW * 4096,), jnp.int32),
        mesh=mesh,
        compiler_params=pltpu.CompilerParams(needs_layout_passes=False),
        scratch_types=[
            pltpu.VMEM((_CHUNK,), jnp.float32),
            pltpu.VMEM((L * 4096,), jnp.int32),
            pltpu.VMEM((4096,), jnp.int32),
        ],
    )
    def pass1(x_hbm, h1_hbm, buf, subhist, merged):
        wid = _wid()
        _zero(subhist, L * 4096)

        def bm(ku):
            b = lax.convert_element_type(ku >> jnp.uint32(20), jnp.int32)
            return b, None, 4096

        _hist_stream(x_hbm, buf, subhist, wid, bm)
        _lane_merge(subhist, merged, 4096)
        pltpu.sync_copy(merged, h1_hbm.at[pl.ds(wid * 4096, 4096)])

    @functools.partial(
        pl.kernel,
        out_type=jax.ShapeDtypeStruct((NW * 4096,), jnp.int32),
        mesh=mesh,
        compiler_params=pltpu.CompilerParams(needs_layout_passes=False),
        scratch_types=[
            pltpu.VMEM((_CHUNK,), jnp.float32),
            pltpu.VMEM((16384,), jnp.int32),
            pltpu.VMEM((L * 4096,), jnp.int32),
            pltpu.VMEM((4096,), jnp.int32),
            pltpu.VMEM((256,), jnp.int32),
        ],
    )
    def pass2(x_hbm, h1_hbm, h2_hbm, buf, tmp, subhist, mg, gsum):
        wid = _wid()
        _global_merge(h1_hbm, tmp, mg)
        hb1, _ = _desc_select(mg, gsum, 4096, jnp.int32(_KK))
        hb1u = lax.convert_element_type(hb1, jnp.uint32)
        _zero(subhist, L * 4096)

        def bm(ku):
            sel = (ku >> jnp.uint32(20)) == hb1u
            b = lax.convert_element_type(
                (ku >> jnp.uint32(8)) & jnp.uint32(0xFFF), jnp.int32)
            return b, sel, 4096

        _hist_stream(x_hbm, buf, subhist, wid, bm)
        _lane_merge(subhist, mg, 4096)
        pltpu.sync_copy(mg, h2_hbm.at[pl.ds(wid * 4096, 4096)])

    @functools.partial(
        pl.kernel,
        out_type=jax.ShapeDtypeStruct((NW * 256,), jnp.int32),
        mesh=mesh,
        compiler_params=pltpu.CompilerParams(needs_layout_passes=False),
        scratch_types=[
            pltpu.VMEM((_CHUNK,), jnp.float32),
            pltpu.VMEM((16384,), jnp.int32),
            pltpu.VMEM((L * 256,), jnp.int32),
            pltpu.VMEM((4096,), jnp.int32),
            pltpu.VMEM((256,), jnp.int32),
        ],
    )
    def pass3(x_hbm, h1_hbm, h2_hbm, h3_hbm, buf, tmp, subhist, mg, gsum):
        wid = _wid()
        _global_merge(h1_hbm, tmp, mg)
        hb1, ab1 = _desc_select(mg, gsum, 4096, jnp.int32(_KK))
        _global_merge(h2_hbm, tmp, mg)
        hb2, _ = _desc_select(mg, gsum, 4096, jnp.int32(_KK) - ab1)
        pref = lax.convert_element_type(hb1 * 4096 + hb2, jnp.uint32)
        _zero(subhist, L * 256)

        def bm(ku):
            sel = (ku >> jnp.uint32(8)) == pref
            b = lax.convert_element_type(ku & jnp.uint32(0xFF), jnp.int32)
            return b, sel, 256

        _hist_stream(x_hbm, buf, subhist, wid, bm)
        _lane_merge(subhist, mg, 256)
        pltpu.sync_copy(mg.at[pl.ds(0, 256)], h3_hbm.at[pl.ds(wid * 256, 256)])

    @functools.partial(
        pl.kernel,
        out_type=jax.ShapeDtypeStruct((_N,), jnp.float32),
        mesh=mesh,
        compiler_params=pltpu.CompilerParams(needs_layout_passes=False),
        scratch_types=[
            pltpu.VMEM((_CHUNK,), jnp.float32),
            pltpu.VMEM((_CHUNK,), jnp.float32),
            pltpu.VMEM((16384,), jnp.int32),
            pltpu.VMEM((4096,), jnp.int32),
            pltpu.VMEM((256,), jnp.int32),
            pltpu.VMEM((256,), jnp.int32),
        ],
    )
    def pass4(x_hbm, h1_hbm, h2_hbm, h3_hbm, y_hbm, bin_, bout, tmp, mg, m3,
              gsum):
        wid = _wid()
        li = _lanes()

        _global_merge(h1_hbm, tmp, mg)
        hb1, ab1 = _desc_select(mg, gsum, 4096, jnp.int32(_KK))
        _global_merge(h2_hbm, tmp, mg)
        hb2, ab2 = _desc_select(mg, gsum, 4096, jnp.int32(_KK) - ab1)

        pltpu.sync_copy(h3_hbm, tmp.at[pl.ds(0, NW * 256)])

        def b3(g, _):
            acc = jnp.zeros((L,), jnp.int32)
            for w in range(NW):
                acc = acc + tmp[pl.ds(w * 256 + g * L, L)]
            m3[pl.ds(g * L, L)] = acc
            return 0

        lax.fori_loop(0, 256 // L, b3, 0)
        kkt3 = jnp.int32(_KK) - ab1 - ab2
        hb3, ab3 = _desc_select(m3, gsum, 256, kkt3)

        t = ((lax.convert_element_type(hb1, jnp.uint32) << jnp.uint32(20))
             | (lax.convert_element_type(hb2, jnp.uint32) << jnp.uint32(8))
             | lax.convert_element_type(hb3, jnp.uint32))
        n_keep_ties = kkt3 - ab3

        cw_lo = plsc.load_gather(tmp, [li * jnp.int32(256) + hb3])
        cw_hi = plsc.load_gather(
            tmp, [(li + jnp.int32(16)) * jnp.int32(256) + hb3])
        cs_lo = plsc.cumsum(cw_lo)
        cs_hi = plsc.cumsum(cw_hi) + _scal(cs_lo)
        my_cw = jnp.where(wid < 16, _at(cw_lo, wid), _at(cw_hi, wid - 16))
        my_incl = jnp.where(wid < 16, _at(cs_lo, wid), _at(cs_hi, wid - 16))
        before_w = my_incl - my_cw
        budget = jnp.clip(n_keep_ties - before_w, 0, my_cw)

        zf = jnp.zeros((L,), jnp.float32)

        def stream_simple(strict):
            def go():
                for c in range(_NCHUNKS):
                    base = wid * _PER_W + c * _CHUNK
                    pltpu.sync_copy(x_hbm.at[pl.ds(base, _CHUNK)], bin_)

                    def inner(j, _):
                        v = bin_[pl.ds(j * L, L)]
                        ku = _ku16(v)
                        keep = ku > t if strict else ku >= t
                        bout[pl.ds(j * L, L)] = jnp.where(keep, v, zf)
                        return 0

                    lax.fori_loop(0, _CHUNK // L, inner, 0, unroll=8)
                    pltpu.sync_copy(bout, y_hbm.at[pl.ds(base, _CHUNK)])

            return go

        def stream_partial():
            one_i = jnp.ones((L,), jnp.int32)
            zero_i = jnp.zeros((L,), jnp.int32)

            def outer(c, r):
                base = wid * _PER_W + c * _CHUNK
                pltpu.sync_copy(x_hbm.at[pl.ds(base, _CHUNK)], bin_)

                def inner(j, rr):
                    v = bin_[pl.ds(j * L, L)]
                    ku = _ku16(v)
                    tie = ku == t
                    cs = plsc.cumsum(jnp.where(tie, one_i, zero_i))
                    keep = (ku > t) | (tie & ((rr + cs) <= budget))
                    bout[pl.ds(j * L, L)] = jnp.where(keep, v, zf)
                    return rr + _scal(cs)

                r = lax.fori_loop(0, _CHUNK // L, inner, r)
                pltpu.sync_copy(bout, y_hbm.at[pl.ds(base, _CHUNK)])
                return r

            lax.fori_loop(0, _NCHUNKS, outer, jnp.int32(0))

        full = budget == my_cw
        none_ = jnp.logical_and(jnp.logical_not(full), budget == 0)
        part = jnp.logical_and(jnp.logical_not(full), budget > 0)

        pl.when(full)(stream_simple(False))
        pl.when(none_)(stream_simple(True))
        pl.when(part)(stream_partial)

    return pass1, pass2, pass3, pass4


@jax.jit
def kernel(hidden_preactivation_BH):
    b, h = hidden_preactivation_BH.shape
    pass1, pass2, pass3, pass4 = _build_passes()
    xf = hidden_preactivation_BH.reshape(-1)
    h1 = pass1(xf)
    h2 = pass2(xf, h1)
    h3 = pass3(xf, h1, h2)
    y = pass4(xf, h1, h2, h3)
    return y.reshape(b, h)


# speedup vs baseline: 1.8887x; 1.8887x over previous
import functools

import jax
import jax.numpy as jnp
from jax import lax
from jax.experimental import pallas as pl
from jax.experimental.pallas import tpu as pltpu
from jax.experimental.pallas import tpu_sc as plsc

NC = 2
NS = 16
NW = NC * NS
L = 16

_B = 128
_H = 32768
_N = _B * _H
_PER_W = _N // NW
_CHUNK = 16384
_NCHUNKS = _PER_W // _CHUNK
_ROWS_PER_CHUNK = 1
_KK = 64 * _B
_W4 = 4


def _wid():
    return lax.axis_index("s") * NC + lax.axis_index("c")


def _lanes():
    return lax.iota(jnp.int32, L)


def _ku16(v):
    i = lax.bitcast_convert_type(v, jnp.int32)
    k = i ^ ((i >> 31) & jnp.int32(0x7FFFFFFF))
    return lax.bitcast_convert_type(k, jnp.uint32) ^ jnp.uint32(0x80000000)


def _chunk_rc(wid, c):
    return wid * (_PER_W // _H) + c // (_H // _CHUNK), (c % (_H // _CHUNK)) * _CHUNK


def _zero(ref, nwords):
    z = jnp.zeros((L,), jnp.int32)

    def b(i, _):
        ref[pl.ds(i * L, L)] = z
        return 0

    lax.fori_loop(0, nwords // L, b, 0, unroll=4)


def _scal(v):
    return jnp.max(v)


def _at(v, lane):
    return jnp.sum(jnp.where(_lanes() == lane, v, jnp.zeros_like(v)))


def _pick(v, kkt, running):
    r = lax.rev(v, (0,))
    cs = plsc.cumsum(r)
    m = (running + cs) >= kkt
    lb = _scal(plsc.all_reduce_ffs(m))
    above = running + _at(cs, lb) - _at(r, lb)
    return jnp.int32(15) - lb, above


def _desc_select(mg, gsum, nbuckets, kkt):
    li = _lanes()
    if nbuckets == 4096:
        def bg(g, _):
            acc = jnp.zeros((L,), jnp.int32)
            for l in range(L):
                acc = acc + plsc.load_gather(mg, [(g * L + li) * L + l])
            gsum[pl.ds(g * L, L)] = acc
            return 0

        lax.fori_loop(0, 16, bg, 0)
        ss = jnp.zeros((L,), jnp.int32)
        for l in range(L):
            ss = ss + plsc.load_gather(gsum, [li * L + l])
        s_star, ab0 = _pick(ss, kkt, jnp.int32(0))
        gvec = gsum[pl.ds(s_star * L, L)]
        g_in, ab1 = _pick(gvec, kkt, ab0)
        g_star = s_star * L + g_in
        bvec = mg[pl.ds(g_star * L, L)]
        b_in, ab2 = _pick(bvec, kkt, ab1)
        return g_star * L + b_in, ab2
    else:
        ss = jnp.zeros((L,), jnp.int32)
        for l in range(L):
            ss = ss + plsc.load_gather(mg, [li * L + l])
        g_star, ab0 = _pick(ss, kkt, jnp.int32(0))
        bvec = mg[pl.ds(g_star * L, L)]
        b_in, ab1 = _pick(bvec, kkt, ab0)
        return g_star * L + b_in, ab1


def _global_merge(h_hbm, tmp, mg):
    _zero(mg, 4096)
    for cc in range(8):
        pltpu.sync_copy(h_hbm.at[pl.ds(cc * 16384, 16384)], tmp)

        def b(g, _):
            acc = mg[pl.ds(g * L, L)]
            for w in range(4):
                acc = acc + tmp[pl.ds(w * 4096 + g * L, L)]
            mg[pl.ds(g * L, L)] = acc
            return 0

        lax.fori_loop(0, 256, b, 0, unroll=4)


def _lane_merge(subhist, out_ref, nbuckets):
    def b(g, _):
        acc = jnp.zeros((L,), jnp.int32)
        for l in range(L):
            acc = acc + subhist[pl.ds(l * nbuckets + g * L, L)]
        out_ref[pl.ds(g * L, L)] = acc
        return 0

    lax.fori_loop(0, nbuckets // L, b, 0)


def _stream_in(x_hbm, wid, bufs, sems, process):
    r0, c0 = _chunk_rc(wid, 0)
    h = [None, None]
    h[0] = pltpu.async_copy(x_hbm.at[r0, pl.ds(c0, _CHUNK)], bufs[0], sems[0])
    for c in range(_NCHUNKS):
        b = c % 2
        h[b].wait()
        if c + 1 < _NCHUNKS:
            nb = (c + 1) % 2
            rn, cn = _chunk_rc(wid, c + 1)
            h[nb] = pltpu.async_copy(
                x_hbm.at[rn, pl.ds(cn, _CHUNK)], bufs[nb], sems[nb])
        process(bufs[b], c)


def _hist_stream(x_hbm, bufs, sems, subhist, wid, bucket_and_mask):
    ones = jnp.ones((L,), jnp.int32)
    li = _lanes()

    def process(buf, c):
        def inner(j, _):
            vs = [buf[pl.ds(j * (L * _W4) + m * L, L)] for m in range(_W4)]
            kus = [_ku16(v) for v in vs]
            bmns = [bucket_and_mask(ku) for ku in kus]
            for bkt, msk, nb in bmns:
                idx = li * jnp.int32(nb) + bkt
                if msk is None:
                    plsc.addupdate_scatter(subhist, [idx], ones)
                else:
                    plsc.addupdate_scatter(subhist, [idx], ones, mask=msk)
            return 0

        lax.fori_loop(0, _CHUNK // (L * _W4), inner, 0, unroll=2)

    _stream_in(x_hbm, wid, bufs, sems, process)


@functools.cache
def _build_passes():
    mesh = plsc.VectorSubcoreMesh(core_axis_name="c", subcore_axis_name="s")
    cp = pltpu.CompilerParams(needs_layout_passes=False)

    @functools.partial(
        pl.kernel,
        out_type=jax.ShapeDtypeStruct((NW * 4096,), jnp.int32),
        mesh=mesh,
        compiler_params=cp,
        scratch_types=[
            pltpu.VMEM((_CHUNK,), jnp.float32),
            pltpu.VMEM((_CHUNK,), jnp.float32),
            pltpu.VMEM((L * 4096,), jnp.int32),
            pltpu.VMEM((4096,), jnp.int32),
            pltpu.SemaphoreType.DMA,
            pltpu.SemaphoreType.DMA,
        ],
    )
    def pass1(x_hbm, h1_hbm, buf0, buf1, subhist, merged, sem0, sem1):
        wid = _wid()
        _zero(subhist, L * 4096)

        def bm(ku):
            b = lax.convert_element_type(ku >> jnp.uint32(20), jnp.int32)
            return b, None, 4096

        _hist_stream(x_hbm, [buf0, buf1], [sem0, sem1], subhist, wid, bm)
        _lane_merge(subhist, merged, 4096)
        pltpu.sync_copy(merged, h1_hbm.at[pl.ds(wid * 4096, 4096)])

    @functools.partial(
        pl.kernel,
        out_type=jax.ShapeDtypeStruct((NW * 4096,), jnp.int32),
        mesh=mesh,
        compiler_params=cp,
        scratch_types=[
            pltpu.VMEM((_CHUNK,), jnp.float32),
            pltpu.VMEM((_CHUNK,), jnp.float32),
            pltpu.VMEM((16384,), jnp.int32),
            pltpu.VMEM((L * 4096,), jnp.int32),
            pltpu.VMEM((4096,), jnp.int32),
            pltpu.VMEM((256,), jnp.int32),
            pltpu.SemaphoreType.DMA,
            pltpu.SemaphoreType.DMA,
        ],
    )
    def pass2(x_hbm, h1_hbm, h2_hbm, buf0, buf1, tmp, subhist, mg, gsum,
              sem0, sem1):
        wid = _wid()
        _global_merge(h1_hbm, tmp, mg)
        hb1, _ = _desc_select(mg, gsum, 4096, jnp.int32(_KK))
        hb1u = lax.convert_element_type(hb1, jnp.uint32)
        _zero(subhist, L * 4096)

        def bm(ku):
            sel = (ku >> jnp.uint32(20)) == hb1u
            b = lax.convert_element_type(
                (ku >> jnp.uint32(8)) & jnp.uint32(0xFFF), jnp.int32)
            return b, sel, 4096

        _hist_stream(x_hbm, [buf0, buf1], [sem0, sem1], subhist, wid, bm)
        _lane_merge(subhist, mg, 4096)
        pltpu.sync_copy(mg, h2_hbm.at[pl.ds(wid * 4096, 4096)])

    @functools.partial(
        pl.kernel,
        out_type=jax.ShapeDtypeStruct((NW * 256,), jnp.int32),
        mesh=mesh,
        compiler_params=cp,
        scratch_types=[
            pltpu.VMEM((_CHUNK,), jnp.float32),
            pltpu.VMEM((_CHUNK,), jnp.float32),
            pltpu.VMEM((16384,), jnp.int32),
            pltpu.VMEM((L * 256,), jnp.int32),
            pltpu.VMEM((4096,), jnp.int32),
            pltpu.VMEM((256,), jnp.int32),
            pltpu.SemaphoreType.DMA,
            pltpu.SemaphoreType.DMA,
        ],
    )
    def pass3(x_hbm, h1_hbm, h2_hbm, h3_hbm, buf0, buf1, tmp, subhist, mg,
              gsum, sem0, sem1):
        wid = _wid()
        _global_merge(h1_hbm, tmp, mg)
        hb1, ab1 = _desc_select(mg, gsum, 4096, jnp.int32(_KK))
        _global_merge(h2_hbm, tmp, mg)
        hb2, _ = _desc_select(mg, gsum, 4096, jnp.int32(_KK) - ab1)
        pref = lax.convert_element_type(hb1 * 4096 + hb2, jnp.uint32)
        _zero(subhist, L * 256)

        def bm(ku):
            sel = (ku >> jnp.uint32(8)) == pref
            b = lax.convert_element_type(ku & jnp.uint32(0xFF), jnp.int32)
            return b, sel, 256

        _hist_stream(x_hbm, [buf0, buf1], [sem0, sem1], subhist, wid, bm)
        _lane_merge(subhist, mg, 256)
        pltpu.sync_copy(mg.at[pl.ds(0, 256)], h3_hbm.at[pl.ds(wid * 256, 256)])

    @functools.partial(
        pl.kernel,
        out_type=jax.ShapeDtypeStruct((_B, _H), jnp.float32),
        mesh=mesh,
        compiler_params=cp,
        scratch_types=[
            pltpu.VMEM((_CHUNK,), jnp.float32),
            pltpu.VMEM((_CHUNK,), jnp.float32),
            pltpu.VMEM((_CHUNK,), jnp.float32),
            pltpu.VMEM((_CHUNK,), jnp.float32),
            pltpu.VMEM((16384,), jnp.int32),
            pltpu.VMEM((4096,), jnp.int32),
            pltpu.VMEM((256,), jnp.int32),
            pltpu.VMEM((256,), jnp.int32),
            pltpu.SemaphoreType.DMA,
            pltpu.SemaphoreType.DMA,
            pltpu.SemaphoreType.DMA,
            pltpu.SemaphoreType.DMA,
        ],
    )
    def pass4(x_hbm, h1_hbm, h2_hbm, h3_hbm, y_hbm, bin0, bin1, bout0, bout1,
              tmp, mg, m3, gsum, rs0, rs1, ws0, ws1):
        wid = _wid()
        li = _lanes()

        _global_merge(h1_hbm, tmp, mg)
        hb1, ab1 = _desc_select(mg, gsum, 4096, jnp.int32(_KK))
        _global_merge(h2_hbm, tmp, mg)
        hb2, ab2 = _desc_select(mg, gsum, 4096, jnp.int32(_KK) - ab1)

        pltpu.sync_copy(h3_hbm, tmp.at[pl.ds(0, NW * 256)])

        def b3(g, _):
            acc = jnp.zeros((L,), jnp.int32)
            for w in range(NW):
                acc = acc + tmp[pl.ds(w * 256 + g * L, L)]
            m3[pl.ds(g * L, L)] = acc
            return 0

        lax.fori_loop(0, 256 // L, b3, 0)
        kkt3 = jnp.int32(_KK) - ab1 - ab2
        hb3, ab3 = _desc_select(m3, gsum, 256, kkt3)

        t = ((lax.convert_element_type(hb1, jnp.uint32) << jnp.uint32(20))
             | (lax.convert_element_type(hb2, jnp.uint32) << jnp.uint32(8))
             | lax.convert_element_type(hb3, jnp.uint32))
        n_keep_ties = kkt3 - ab3

        cw_lo = plsc.load_gather(tmp, [li * jnp.int32(256) + hb3])
        cw_hi = plsc.load_gather(
            tmp, [(li + jnp.int32(16)) * jnp.int32(256) + hb3])
        cs_lo = plsc.cumsum(cw_lo)
        cs_hi = plsc.cumsum(cw_hi) + _scal(cs_lo)
        my_cw = jnp.where(wid < 16, _at(cw_lo, wid), _at(cw_hi, wid - 16))
        my_incl = jnp.where(wid < 16, _at(cs_lo, wid), _at(cs_hi, wid - 16))
        before_w = my_incl - my_cw
        budget = jnp.clip(n_keep_ties - before_w, 0, my_cw)

        zf = jnp.zeros((L,), jnp.float32)
        bouts = [bout0, bout1]
        wsems = [ws0, ws1]

        def stream_simple(strict):
            def go():
                wh = [None, None]

                def process(buf, c):
                    b = c % 2
                    if wh[b] is not None:
                        wh[b].wait()

                    def inner(j, _):
                        vs = [buf[pl.ds(j * (L * _W4) + m * L, L)]
                              for m in range(_W4)]
                        kus = [_ku16(v) for v in vs]
                        for m in range(_W4):
                            keep = kus[m] > t if strict else kus[m] >= t
                            bouts[b][pl.ds(j * (L * _W4) + m * L, L)] = (
                                jnp.where(keep, vs[m], zf))
                        return 0

                    lax.fori_loop(0, _CHUNK // (L * _W4), inner, 0, unroll=2)
                    r, cc = _chunk_rc(wid, c)
                    wh[b] = pltpu.async_copy(
                        bouts[b], y_hbm.at[r, pl.ds(cc, _CHUNK)], wsems[b])

                _stream_in(x_hbm, wid, [bin0, bin1], [rs0, rs1], process)
                for b in range(2):
                    if wh[b] is not None:
                        wh[b].wait()

            return go

        def stream_partial():
            one_i = jnp.ones((L,), jnp.int32)
            zero_i = jnp.zeros((L,), jnp.int32)

            def outer(c, r):
                rr_, cc_ = _chunk_rc(wid, c)
                pltpu.sync_copy(x_hbm.at[rr_, pl.ds(cc_, _CHUNK)], bin0)

                def inner(j, rr):
                    v = bin0[pl.ds(j * L, L)]
                    ku = _ku16(v)
                    tie = ku == t
                    cs = plsc.cumsum(jnp.where(tie, one_i, zero_i))
                    keep = (ku > t) | (tie & ((rr + cs) <= budget))
                    bout0[pl.ds(j * L, L)] = jnp.where(keep, v, zf)
                    return rr + _scal(cs)

                r = lax.fori_loop(0, _CHUNK // L, inner, r)
                pltpu.sync_copy(bout0, y_hbm.at[rr_, pl.ds(cc_, _CHUNK)])
                return r

            lax.fori_loop(0, _NCHUNKS, outer, jnp.int32(0))

        full = budget == my_cw
        none_ = jnp.logical_and(jnp.logical_not(full), budget == 0)
        part = jnp.logical_and(jnp.logical_not(full), budget > 0)

        pl.when(full)(stream_simple(False))
        pl.when(none_)(stream_simple(True))
        pl.when(part)(stream_partial)

    return pass1, pass2, pass3, pass4


@jax.jit
def kernel(hidden_preactivation_BH):
    pass1, pass2, pass3, pass4 = _build_passes()
    x = hidden_preactivation_BH
    h1 = pass1(x)
    h2 = pass2(x, h1)
    h3 = pass3(x, h1, h2)
    return pass4(x, h1, h2, h3)


# speedup vs baseline: 2.5505x; 1.3504x over previous
import functools

import jax
import jax.numpy as jnp
from jax import lax
from jax.experimental import pallas as pl
from jax.experimental.pallas import tpu as pltpu
from jax.experimental.pallas import tpu_sc as plsc

NC = 2
NS = 16
NW = NC * NS
L = 16

_B = 128
_H = 32768
_N = _B * _H
_PER_W = _N // NW
_CHUNK = 16384
_NCHUNKS = _PER_W // _CHUNK
_ROWS_PER_CHUNK = 1
_KK = 64 * _B
_W4 = 4


def _wid():
    return lax.axis_index("s") * NC + lax.axis_index("c")


def _lanes():
    return lax.iota(jnp.int32, L)


def _ku16(v):
    i = lax.bitcast_convert_type(v, jnp.int32)
    k = i ^ ((i >> 31) & jnp.int32(0x7FFFFFFF))
    return lax.bitcast_convert_type(k, jnp.uint32) ^ jnp.uint32(0x80000000)


def _chunk_rc(wid, c):
    return wid * (_PER_W // _H) + c // (_H // _CHUNK), (c % (_H // _CHUNK)) * _CHUNK


def _zero(ref, nwords):
    z = jnp.zeros((L,), jnp.int32)

    def b(i, _):
        ref[pl.ds(i * L, L)] = z
        return 0

    lax.fori_loop(0, nwords // L, b, 0, unroll=4)


def _scal(v):
    return jnp.max(v)


def _at(v, lane):
    return jnp.sum(jnp.where(_lanes() == lane, v, jnp.zeros_like(v)))


def _pick(v, kkt, running):
    r = lax.rev(v, (0,))
    cs = plsc.cumsum(r)
    m = (running + cs) >= kkt
    lb = _scal(plsc.all_reduce_ffs(m))
    above = running + _at(cs, lb) - _at(r, lb)
    return jnp.int32(15) - lb, above


def _desc_select(mg, gsum, nbuckets, kkt):
    li = _lanes()
    if nbuckets == 4096:
        def bg(g, _):
            acc = jnp.zeros((L,), jnp.int32)
            for l in range(L):
                acc = acc + plsc.load_gather(mg, [(g * L + li) * L + l])
            gsum[pl.ds(g * L, L)] = acc
            return 0

        lax.fori_loop(0, 16, bg, 0)
        ss = jnp.zeros((L,), jnp.int32)
        for l in range(L):
            ss = ss + plsc.load_gather(gsum, [li * L + l])
        s_star, ab0 = _pick(ss, kkt, jnp.int32(0))
        gvec = gsum[pl.ds(s_star * L, L)]
        g_in, ab1 = _pick(gvec, kkt, ab0)
        g_star = s_star * L + g_in
        bvec = mg[pl.ds(g_star * L, L)]
        b_in, ab2 = _pick(bvec, kkt, ab1)
        return g_star * L + b_in, ab2
    else:
        ss = jnp.zeros((L,), jnp.int32)
        for l in range(L):
            ss = ss + plsc.load_gather(mg, [li * L + l])
        g_star, ab0 = _pick(ss, kkt, jnp.int32(0))
        bvec = mg[pl.ds(g_star * L, L)]
        b_in, ab1 = _pick(bvec, kkt, ab0)
        return g_star * L + b_in, ab1


def _global_merge(h_hbm, tmp, mg):
    _zero(mg, 4096)
    for cc in range(8):
        pltpu.sync_copy(h_hbm.at[pl.ds(cc * 16384, 16384)], tmp)

        def b(g, _):
            acc = mg[pl.ds(g * L, L)]
            for w in range(4):
                acc = acc + tmp[pl.ds(w * 4096 + g * L, L)]
            mg[pl.ds(g * L, L)] = acc
            return 0

        lax.fori_loop(0, 256, b, 0, unroll=4)


def _lane_merge(subhist, out_ref, nbuckets):
    def b(g, _):
        acc = jnp.zeros((L,), jnp.int32)
        for l in range(L):
            acc = acc + subhist[pl.ds(l * nbuckets + g * L, L)]
        out_ref[pl.ds(g * L, L)] = acc
        return 0

    lax.fori_loop(0, nbuckets // L, b, 0)


def _stream_in(x_hbm, wid, bufs, sems, process):
    r0, c0 = _chunk_rc(wid, 0)
    h = [None, None]
    h[0] = pltpu.async_copy(x_hbm.at[r0, pl.ds(c0, _CHUNK)], bufs[0], sems[0])
    for c in range(_NCHUNKS):
        b = c % 2
        h[b].wait()
        if c + 1 < _NCHUNKS:
            nb = (c + 1) % 2
            rn, cn = _chunk_rc(wid, c + 1)
            h[nb] = pltpu.async_copy(
                x_hbm.at[rn, pl.ds(cn, _CHUNK)], bufs[nb], sems[nb])
        process(bufs[b], c)


def _hist_stream(x_hbm, bufs, sems, subhist, wid, bucket_and_mask):
    ones = jnp.ones((L,), jnp.int32)
    li = _lanes()

    def process(buf, c):
        def inner(j, _):
            vs = [buf[pl.ds(j * (L * _W4) + m * L, L)] for m in range(_W4)]
            kus = [_ku16(v) for v in vs]
            bmns = [bucket_and_mask(ku) for ku in kus]
            for bkt, msk, nb in bmns:
                idx = li * jnp.int32(nb) + bkt
                if msk is None:
                    plsc.addupdate_scatter(subhist, [idx], ones)
                else:
                    plsc.addupdate_scatter(subhist, [idx], ones, mask=msk)
            return 0

        lax.fori_loop(0, _CHUNK // (L * _W4), inner, 0, unroll=2)

    _stream_in(x_hbm, wid, bufs, sems, process)


@functools.cache
def _build_passes():
    mesh = plsc.VectorSubcoreMesh(core_axis_name="c", subcore_axis_name="s")
    cp = pltpu.CompilerParams(needs_layout_passes=False)

    @functools.partial(
        pl.kernel,
        out_type=jax.ShapeDtypeStruct((NW * 4096,), jnp.int32),
        mesh=mesh,
        compiler_params=cp,
        scratch_types=[
            pltpu.VMEM((_CHUNK,), jnp.float32),
            pltpu.VMEM((_CHUNK,), jnp.float32),
            pltpu.VMEM((L * 4096,), jnp.int32),
            pltpu.VMEM((4096,), jnp.int32),
            pltpu.SemaphoreType.DMA,
            pltpu.SemaphoreType.DMA,
        ],
    )
    def pass1(x_hbm, h1_hbm, buf0, buf1, subhist, merged, sem0, sem1):
        wid = _wid()
        _zero(subhist, L * 4096)

        def bm(ku):
            b = lax.convert_element_type(ku >> jnp.uint32(20), jnp.int32)
            return b, None, 4096

        _hist_stream(x_hbm, [buf0, buf1], [sem0, sem1], subhist, wid, bm)
        _lane_merge(subhist, merged, 4096)
        pltpu.sync_copy(merged, h1_hbm.at[pl.ds(wid * 4096, 4096)])

    @functools.partial(
        pl.kernel,
        out_type=[jax.ShapeDtypeStruct((NW * 4096,), jnp.int32),
                  jax.ShapeDtypeStruct((L,), jnp.int32)],
        mesh=mesh,
        compiler_params=cp,
        scratch_types=[
            pltpu.VMEM((_CHUNK,), jnp.float32),
            pltpu.VMEM((_CHUNK,), jnp.float32),
            pltpu.VMEM((16384,), jnp.int32),
            pltpu.VMEM((L * 4096,), jnp.int32),
            pltpu.VMEM((4096,), jnp.int32),
            pltpu.VMEM((256,), jnp.int32),
            pltpu.VMEM((L,), jnp.int32),
            pltpu.SemaphoreType.DMA,
            pltpu.SemaphoreType.DMA,
        ],
    )
    def pass2(x_hbm, h1_hbm, h2_hbm, sel1_hbm, buf0, buf1, tmp, subhist, mg,
              gsum, selbuf, sem0, sem1):
        wid = _wid()
        li = _lanes()
        _global_merge(h1_hbm, tmp, mg)
        hb1, ab1 = _desc_select(mg, gsum, 4096, jnp.int32(_KK))
        hb1u = lax.convert_element_type(hb1, jnp.uint32)

        @pl.when(wid == 0)
        def _():
            z = jnp.zeros((L,), jnp.int32)
            sv = jnp.where(li == 0, hb1, jnp.where(li == 1, ab1, z))
            selbuf[pl.ds(0, L)] = sv
            pltpu.sync_copy(selbuf, sel1_hbm)
        _zero(subhist, L * 4096)

        def bm(ku):
            sel = (ku >> jnp.uint32(20)) == hb1u
            b = lax.convert_element_type(
                (ku >> jnp.uint32(8)) & jnp.uint32(0xFFF), jnp.int32)
            return b, sel, 4096

        _hist_stream(x_hbm, [buf0, buf1], [sem0, sem1], subhist, wid, bm)
        _lane_merge(subhist, mg, 4096)
        pltpu.sync_copy(mg, h2_hbm.at[pl.ds(wid * 4096, 4096)])

    @functools.partial(
        pl.kernel,
        out_type=[jax.ShapeDtypeStruct((NW * 256,), jnp.int32),
                  jax.ShapeDtypeStruct((L,), jnp.int32)],
        mesh=mesh,
        compiler_params=cp,
        scratch_types=[
            pltpu.VMEM((_CHUNK,), jnp.float32),
            pltpu.VMEM((_CHUNK,), jnp.float32),
            pltpu.VMEM((16384,), jnp.int32),
            pltpu.VMEM((L * 256,), jnp.int32),
            pltpu.VMEM((4096,), jnp.int32),
            pltpu.VMEM((256,), jnp.int32),
            pltpu.VMEM((L,), jnp.int32),
            pltpu.SemaphoreType.DMA,
            pltpu.SemaphoreType.DMA,
        ],
    )
    def pass3(x_hbm, h2_hbm, sel1_hbm, h3_hbm, sel2_hbm, buf0, buf1, tmp,
              subhist, mg, gsum, selbuf, sem0, sem1):
        wid = _wid()
        li = _lanes()
        pltpu.sync_copy(sel1_hbm, selbuf)
        sv1 = selbuf[pl.ds(0, L)]
        hb1 = _at(sv1, 0)
        ab1 = _at(sv1, 1)
        _global_merge(h2_hbm, tmp, mg)
        hb2, ab2 = _desc_select(mg, gsum, 4096, jnp.int32(_KK) - ab1)
        pref = lax.convert_element_type(hb1 * 4096 + hb2, jnp.uint32)

        @pl.when(wid == 0)
        def _():
            z = jnp.zeros((L,), jnp.int32)
            sv = jnp.where(li == 0, hb1,
                           jnp.where(li == 1, ab1,
                                     jnp.where(li == 2, hb2,
                                               jnp.where(li == 3, ab2, z))))
            selbuf[pl.ds(0, L)] = sv
            pltpu.sync_copy(selbuf, sel2_hbm)
        _zero(subhist, L * 256)

        def bm(ku):
            sel = (ku >> jnp.uint32(8)) == pref
            b = lax.convert_element_type(ku & jnp.uint32(0xFF), jnp.int32)
            return b, sel, 256

        _hist_stream(x_hbm, [buf0, buf1], [sem0, sem1], subhist, wid, bm)
        _lane_merge(subhist, mg, 256)
        pltpu.sync_copy(mg.at[pl.ds(0, 256)], h3_hbm.at[pl.ds(wid * 256, 256)])

    @functools.partial(
        pl.kernel,
        out_type=jax.ShapeDtypeStruct((_B, _H), jnp.float32),
        mesh=mesh,
        compiler_params=cp,
        scratch_types=[
            pltpu.VMEM((_CHUNK,), jnp.float32),
            pltpu.VMEM((_CHUNK,), jnp.float32),
            pltpu.VMEM((_CHUNK,), jnp.float32),
            pltpu.VMEM((_CHUNK,), jnp.float32),
            pltpu.VMEM((16384,), jnp.int32),
            pltpu.VMEM((4096,), jnp.int32),
            pltpu.VMEM((256,), jnp.int32),
            pltpu.VMEM((256,), jnp.int32),
            pltpu.SemaphoreType.DMA,
            pltpu.SemaphoreType.DMA,
            pltpu.SemaphoreType.DMA,
            pltpu.SemaphoreType.DMA,
        ],
    )
    def pass4(x_hbm, h3_hbm, sel2_hbm, y_hbm, bin0, bin1, bout0, bout1,
              tmp, mg, m3, gsum, rs0, rs1, ws0, ws1):
        wid = _wid()
        li = _lanes()

        pltpu.sync_copy(sel2_hbm, mg.at[pl.ds(0, L)])
        sv2 = mg[pl.ds(0, L)]
        hb1 = _at(sv2, 0)
        ab1 = _at(sv2, 1)
        hb2 = _at(sv2, 2)
        ab2 = _at(sv2, 3)

        pltpu.sync_copy(h3_hbm, tmp.at[pl.ds(0, NW * 256)])

        def b3(g, _):
            acc = jnp.zeros((L,), jnp.int32)
            for w in range(NW):
                acc = acc + tmp[pl.ds(w * 256 + g * L, L)]
            m3[pl.ds(g * L, L)] = acc
            return 0

        lax.fori_loop(0, 256 // L, b3, 0)
        kkt3 = jnp.int32(_KK) - ab1 - ab2
        hb3, ab3 = _desc_select(m3, gsum, 256, kkt3)

        t = ((lax.convert_element_type(hb1, jnp.uint32) << jnp.uint32(20))
             | (lax.convert_element_type(hb2, jnp.uint32) << jnp.uint32(8))
             | lax.convert_element_type(hb3, jnp.uint32))
        n_keep_ties = kkt3 - ab3

        cw_lo = plsc.load_gather(tmp, [li * jnp.int32(256) + hb3])
        cw_hi = plsc.load_gather(
            tmp, [(li + jnp.int32(16)) * jnp.int32(256) + hb3])
        cs_lo = plsc.cumsum(cw_lo)
        cs_hi = plsc.cumsum(cw_hi) + _scal(cs_lo)
        my_cw = jnp.where(wid < 16, _at(cw_lo, wid), _at(cw_hi, wid - 16))
        my_incl = jnp.where(wid < 16, _at(cs_lo, wid), _at(cs_hi, wid - 16))
        before_w = my_incl - my_cw
        budget = jnp.clip(n_keep_ties - before_w, 0, my_cw)

        zf = jnp.zeros((L,), jnp.float32)
        bouts = [bout0, bout1]
        wsems = [ws0, ws1]

        def stream_simple(strict):
            def go():
                wh = [None, None]

                def process(buf, c):
                    b = c % 2
                    if wh[b] is not None:
                        wh[b].wait()

                    def inner(j, _):
                        vs = [buf[pl.ds(j * (L * _W4) + m * L, L)]
                              for m in range(_W4)]
                        kus = [_ku16(v) for v in vs]
                        for m in range(_W4):
                            keep = kus[m] > t if strict else kus[m] >= t
                            bouts[b][pl.ds(j * (L * _W4) + m * L, L)] = (
                                jnp.where(keep, vs[m], zf))
                        return 0

                    lax.fori_loop(0, _CHUNK // (L * _W4), inner, 0, unroll=2)
                    r, cc = _chunk_rc(wid, c)
                    wh[b] = pltpu.async_copy(
                        bouts[b], y_hbm.at[r, pl.ds(cc, _CHUNK)], wsems[b])

                _stream_in(x_hbm, wid, [bin0, bin1], [rs0, rs1], process)
                for b in range(2):
                    if wh[b] is not None:
                        wh[b].wait()

            return go

        def stream_partial():
            one_i = jnp.ones((L,), jnp.int32)
            zero_i = jnp.zeros((L,), jnp.int32)

            def outer(c, r):
                rr_, cc_ = _chunk_rc(wid, c)
                pltpu.sync_copy(x_hbm.at[rr_, pl.ds(cc_, _CHUNK)], bin0)

                def inner(j, rr):
                    v = bin0[pl.ds(j * L, L)]
                    ku = _ku16(v)
                    tie = ku == t
                    cs = plsc.cumsum(jnp.where(tie, one_i, zero_i))
                    keep = (ku > t) | (tie & ((rr + cs) <= budget))
                    bout0[pl.ds(j * L, L)] = jnp.where(keep, v, zf)
                    return rr + _scal(cs)

                r = lax.fori_loop(0, _CHUNK // L, inner, r)
                pltpu.sync_copy(bout0, y_hbm.at[rr_, pl.ds(cc_, _CHUNK)])
                return r

            lax.fori_loop(0, _NCHUNKS, outer, jnp.int32(0))

        full = budget == my_cw
        none_ = jnp.logical_and(jnp.logical_not(full), budget == 0)
        part = jnp.logical_and(jnp.logical_not(full), budget > 0)

        pl.when(full)(stream_simple(False))
        pl.when(none_)(stream_simple(True))
        pl.when(part)(stream_partial)

    return pass1, pass2, pass3, pass4


@jax.jit
def kernel(hidden_preactivation_BH):
    pass1, pass2, pass3, pass4 = _build_passes()
    x = hidden_preactivation_BH
    h1 = pass1(x)
    h2, sel1 = pass2(x, h1)
    h3, sel2 = pass3(x, h2, sel1)
    return pass4(x, h3, sel2)


# speedup vs baseline: 3.1398x; 1.2311x over previous
import functools

import jax
import jax.numpy as jnp
from jax import lax
from jax.experimental import pallas as pl
from jax.experimental.pallas import tpu as pltpu
from jax.experimental.pallas import tpu_sc as plsc

NC = 2
NS = 16
NW = NC * NS
L = 16

_B = 128
_H = 32768
_N = _B * _H
_PER_W = _N // NW
_CHUNK = 16384
_NCHUNKS = _PER_W // _CHUNK
_ROWS_PER_CHUNK = 1
_KK = 64 * _B
_W4 = 4


def _wid():
    return lax.axis_index("s") * NC + lax.axis_index("c")


def _lanes():
    return lax.iota(jnp.int32, L)


def _ku16(v):
    i = lax.bitcast_convert_type(v, jnp.int32)
    k = i ^ ((i >> 31) & jnp.int32(0x7FFFFFFF))
    return lax.bitcast_convert_type(k, jnp.uint32) ^ jnp.uint32(0x80000000)


def _chunk_rc(wid, c):
    return wid * (_PER_W // _H) + c // (_H // _CHUNK), (c % (_H // _CHUNK)) * _CHUNK


def _zero(ref, nwords):
    z = jnp.zeros((L,), jnp.int32)

    def b(i, _):
        ref[pl.ds(i * L, L)] = z
        return 0

    lax.fori_loop(0, nwords // L, b, 0, unroll=4)


def _scal(v):
    return jnp.max(v)


def _at(v, lane):
    return jnp.sum(jnp.where(_lanes() == lane, v, jnp.zeros_like(v)))


def _pick(v, kkt, running):
    r = lax.rev(v, (0,))
    cs = plsc.cumsum(r)
    m = (running + cs) >= kkt
    lb = _scal(plsc.all_reduce_ffs(m))
    above = running + _at(cs, lb) - _at(r, lb)
    return jnp.int32(15) - lb, above


def _desc_select(mg, gsum, nbuckets, kkt):
    li = _lanes()
    if nbuckets == 4096:
        def bg(g, _):
            acc = jnp.zeros((L,), jnp.int32)
            for l in range(L):
                acc = acc + plsc.load_gather(mg, [(g * L + li) * L + l])
            gsum[pl.ds(g * L, L)] = acc
            return 0

        lax.fori_loop(0, 16, bg, 0)
        ss = jnp.zeros((L,), jnp.int32)
        for l in range(L):
            ss = ss + plsc.load_gather(gsum, [li * L + l])
        s_star, ab0 = _pick(ss, kkt, jnp.int32(0))
        gvec = gsum[pl.ds(s_star * L, L)]
        g_in, ab1 = _pick(gvec, kkt, ab0)
        g_star = s_star * L + g_in
        bvec = mg[pl.ds(g_star * L, L)]
        b_in, ab2 = _pick(bvec, kkt, ab1)
        return g_star * L + b_in, ab2
    else:
        ss = jnp.zeros((L,), jnp.int32)
        for l in range(L):
            ss = ss + plsc.load_gather(mg, [li * L + l])
        g_star, ab0 = _pick(ss, kkt, jnp.int32(0))
        bvec = mg[pl.ds(g_star * L, L)]
        b_in, ab1 = _pick(bvec, kkt, ab0)
        return g_star * L + b_in, ab1


def _global_merge(h_hbm, tmp, mg):
    _zero(mg, 4096)
    for cc in range(8):
        pltpu.sync_copy(h_hbm.at[pl.ds(cc * 16384, 16384)], tmp)

        def b(g, _):
            acc = mg[pl.ds(g * L, L)]
            for w in range(4):
                acc = acc + tmp[pl.ds(w * 4096 + g * L, L)]
            mg[pl.ds(g * L, L)] = acc
            return 0

        lax.fori_loop(0, 256, b, 0, unroll=4)


def _sc_merge_publish(merged, subhist, shm, hsc_hbm):
    sid = lax.axis_index("s")
    cid = lax.axis_index("c")
    pltpu.sync_copy(merged, shm.at[pl.ds(sid * 4096, 4096)])
    plsc.subcore_barrier()
    for r in range(NS):
        pltpu.sync_copy(shm.at[pl.ds(r * 4096 + sid * 256, 256)],
                        subhist.at[pl.ds(r * 256, 256)])

    def rb(g, _):
        acc = jnp.zeros((L,), jnp.int32)
        for r in range(NS):
            acc = acc + subhist[pl.ds(r * 256 + g * L, L)]
        merged[pl.ds(g * L, L)] = acc
        return 0

    lax.fori_loop(0, 16, rb, 0)
    pltpu.sync_copy(merged.at[pl.ds(0, 256)],
                    hsc_hbm.at[pl.ds(cid * 4096 + sid * 256, 256)])


def _global_merge2(hsc_hbm, tmp, mg):
    pltpu.sync_copy(hsc_hbm, tmp.at[pl.ds(0, 2 * 4096)])

    def b(g, _):
        mg[pl.ds(g * L, L)] = (tmp[pl.ds(g * L, L)]
                               + tmp[pl.ds(4096 + g * L, L)])
        return 0

    lax.fori_loop(0, 256, b, 0, unroll=4)


def _lane_merge(subhist, out_ref, nbuckets):
    def b(g, _):
        acc = jnp.zeros((L,), jnp.int32)
        for l in range(L):
            acc = acc + subhist[pl.ds(l * nbuckets + g * L, L)]
        out_ref[pl.ds(g * L, L)] = acc
        return 0

    lax.fori_loop(0, nbuckets // L, b, 0)


def _stream_in(x_hbm, wid, bufs, sems, process):
    r0, c0 = _chunk_rc(wid, 0)
    h = [None, None]
    h[0] = pltpu.async_copy(x_hbm.at[r0, pl.ds(c0, _CHUNK)], bufs[0], sems[0])
    for c in range(_NCHUNKS):
        b = c % 2
        h[b].wait()
        if c + 1 < _NCHUNKS:
            nb = (c + 1) % 2
            rn, cn = _chunk_rc(wid, c + 1)
            h[nb] = pltpu.async_copy(
                x_hbm.at[rn, pl.ds(cn, _CHUNK)], bufs[nb], sems[nb])
        process(bufs[b], c)


def _hist_stream(x_hbm, bufs, sems, subhist, wid, bucket_and_mask):
    ones = jnp.ones((L,), jnp.int32)
    li = _lanes()

    def process(buf, c):
        def inner(j, _):
            vs = [buf[pl.ds(j * (L * _W4) + m * L, L)] for m in range(_W4)]
            kus = [_ku16(v) for v in vs]
            bmns = [bucket_and_mask(ku) for ku in kus]
            for bkt, msk, nb in bmns:
                idx = li * jnp.int32(nb) + bkt
                if msk is None:
                    plsc.addupdate_scatter(subhist, [idx], ones)
                else:
                    plsc.addupdate_scatter(subhist, [idx], ones, mask=msk)
            return 0

        lax.fori_loop(0, _CHUNK // (L * _W4), inner, 0, unroll=2)

    _stream_in(x_hbm, wid, bufs, sems, process)


@functools.cache
def _build_passes():
    mesh = plsc.VectorSubcoreMesh(core_axis_name="c", subcore_axis_name="s")
    cp = pltpu.CompilerParams(needs_layout_passes=False)

    @functools.partial(
        pl.kernel,
        out_type=jax.ShapeDtypeStruct((NC * 4096,), jnp.int32),
        mesh=mesh,
        compiler_params=cp,
        scratch_types=[
            pltpu.VMEM((_CHUNK,), jnp.float32),
            pltpu.VMEM((_CHUNK,), jnp.float32),
            pltpu.VMEM((L * 4096,), jnp.int32),
            pltpu.VMEM((4096,), jnp.int32),
            pltpu.VMEM_SHARED((NS * 4096,), jnp.int32),
            pltpu.SemaphoreType.DMA,
            pltpu.SemaphoreType.DMA,
        ],
    )
    def pass1(x_hbm, h1_hbm, buf0, buf1, subhist, merged, shm, sem0, sem1):
        wid = _wid()
        _zero(subhist, L * 4096)

        def bm(ku):
            b = lax.convert_element_type(ku >> jnp.uint32(20), jnp.int32)
            return b, None, 4096

        _hist_stream(x_hbm, [buf0, buf1], [sem0, sem1], subhist, wid, bm)
        _lane_merge(subhist, merged, 4096)
        _sc_merge_publish(merged, subhist, shm, h1_hbm)

    @functools.partial(
        pl.kernel,
        out_type=[jax.ShapeDtypeStruct((NC * 4096,), jnp.int32),
                  jax.ShapeDtypeStruct((L,), jnp.int32)],
        mesh=mesh,
        compiler_params=cp,
        scratch_types=[
            pltpu.VMEM((_CHUNK,), jnp.float32),
            pltpu.VMEM((_CHUNK,), jnp.float32),
            pltpu.VMEM((16384,), jnp.int32),
            pltpu.VMEM((L * 4096,), jnp.int32),
            pltpu.VMEM((4096,), jnp.int32),
            pltpu.VMEM((256,), jnp.int32),
            pltpu.VMEM((L,), jnp.int32),
            pltpu.VMEM_SHARED((NS * 4096,), jnp.int32),
            pltpu.SemaphoreType.DMA,
            pltpu.SemaphoreType.DMA,
        ],
    )
    def pass2(x_hbm, h1_hbm, h2_hbm, sel1_hbm, buf0, buf1, tmp, subhist, mg,
              gsum, selbuf, shm, sem0, sem1):
        wid = _wid()
        li = _lanes()
        _global_merge2(h1_hbm, tmp, mg)
        hb1, ab1 = _desc_select(mg, gsum, 4096, jnp.int32(_KK))
        hb1u = lax.convert_element_type(hb1, jnp.uint32)

        @pl.when(wid == 0)
        def _():
            z = jnp.zeros((L,), jnp.int32)
            sv = jnp.where(li == 0, hb1, jnp.where(li == 1, ab1, z))
            selbuf[pl.ds(0, L)] = sv
            pltpu.sync_copy(selbuf, sel1_hbm)
        _zero(subhist, L * 4096)

        def bm(ku):
            sel = (ku >> jnp.uint32(20)) == hb1u
            b = lax.convert_element_type(
                (ku >> jnp.uint32(8)) & jnp.uint32(0xFFF), jnp.int32)
            return b, sel, 4096

        _hist_stream(x_hbm, [buf0, buf1], [sem0, sem1], subhist, wid, bm)
        _lane_merge(subhist, mg, 4096)
        _sc_merge_publish(mg, subhist, shm, h2_hbm)

    @functools.partial(
        pl.kernel,
        out_type=[jax.ShapeDtypeStruct((NW * 256,), jnp.int32),
                  jax.ShapeDtypeStruct((L,), jnp.int32)],
        mesh=mesh,
        compiler_params=cp,
        scratch_types=[
            pltpu.VMEM((_CHUNK,), jnp.float32),
            pltpu.VMEM((_CHUNK,), jnp.float32),
            pltpu.VMEM((16384,), jnp.int32),
            pltpu.VMEM((L * 256,), jnp.int32),
            pltpu.VMEM((4096,), jnp.int32),
            pltpu.VMEM((256,), jnp.int32),
            pltpu.VMEM((L,), jnp.int32),
            pltpu.SemaphoreType.DMA,
            pltpu.SemaphoreType.DMA,
        ],
    )
    def pass3(x_hbm, h2_hbm, sel1_hbm, h3_hbm, sel2_hbm, buf0, buf1, tmp,
              subhist, mg, gsum, selbuf, sem0, sem1):
        wid = _wid()
        li = _lanes()
        pltpu.sync_copy(sel1_hbm, selbuf)
        sv1 = selbuf[pl.ds(0, L)]
        hb1 = _at(sv1, 0)
        ab1 = _at(sv1, 1)
        _global_merge2(h2_hbm, tmp, mg)
        hb2, ab2 = _desc_select(mg, gsum, 4096, jnp.int32(_KK) - ab1)
        pref = lax.convert_element_type(hb1 * 4096 + hb2, jnp.uint32)

        @pl.when(wid == 0)
        def _():
            z = jnp.zeros((L,), jnp.int32)
            sv = jnp.where(li == 0, hb1,
                           jnp.where(li == 1, ab1,
                                     jnp.where(li == 2, hb2,
                                               jnp.where(li == 3, ab2, z))))
            selbuf[pl.ds(0, L)] = sv
            pltpu.sync_copy(selbuf, sel2_hbm)
        _zero(subhist, L * 256)

        def bm(ku):
            sel = (ku >> jnp.uint32(8)) == pref
            b = lax.convert_element_type(ku & jnp.uint32(0xFF), jnp.int32)
            return b, sel, 256

        _hist_stream(x_hbm, [buf0, buf1], [sem0, sem1], subhist, wid, bm)
        _lane_merge(subhist, mg, 256)
        pltpu.sync_copy(mg.at[pl.ds(0, 256)], h3_hbm.at[pl.ds(wid * 256, 256)])

    @functools.partial(
        pl.kernel,
        out_type=jax.ShapeDtypeStruct((_B, _H), jnp.float32),
        mesh=mesh,
        compiler_params=cp,
        scratch_types=[
            pltpu.VMEM((_CHUNK,), jnp.float32),
            pltpu.VMEM((_CHUNK,), jnp.float32),
            pltpu.VMEM((_CHUNK,), jnp.float32),
            pltpu.VMEM((_CHUNK,), jnp.float32),
            pltpu.VMEM((16384,), jnp.int32),
            pltpu.VMEM((4096,), jnp.int32),
            pltpu.VMEM((256,), jnp.int32),
            pltpu.VMEM((256,), jnp.int32),
            pltpu.SemaphoreType.DMA,
            pltpu.SemaphoreType.DMA,
            pltpu.SemaphoreType.DMA,
            pltpu.SemaphoreType.DMA,
        ],
    )
    def pass4(x_hbm, h3_hbm, sel2_hbm, y_hbm, bin0, bin1, bout0, bout1,
              tmp, mg, m3, gsum, rs0, rs1, ws0, ws1):
        wid = _wid()
        li = _lanes()

        pltpu.sync_copy(sel2_hbm, mg.at[pl.ds(0, L)])
        sv2 = mg[pl.ds(0, L)]
        hb1 = _at(sv2, 0)
        ab1 = _at(sv2, 1)
        hb2 = _at(sv2, 2)
        ab2 = _at(sv2, 3)

        pltpu.sync_copy(h3_hbm, tmp.at[pl.ds(0, NW * 256)])

        def b3(g, _):
            acc = jnp.zeros((L,), jnp.int32)
            for w in range(NW):
                acc = acc + tmp[pl.ds(w * 256 + g * L, L)]
            m3[pl.ds(g * L, L)] = acc
            return 0

        lax.fori_loop(0, 256 // L, b3, 0)
        kkt3 = jnp.int32(_KK) - ab1 - ab2
        hb3, ab3 = _desc_select(m3, gsum, 256, kkt3)

        t = ((lax.convert_element_type(hb1, jnp.uint32) << jnp.uint32(20))
             | (lax.convert_element_type(hb2, jnp.uint32) << jnp.uint32(8))
             | lax.convert_element_type(hb3, jnp.uint32))
        n_keep_ties = kkt3 - ab3

        cw_lo = plsc.load_gather(tmp, [li * jnp.int32(256) + hb3])
        cw_hi = plsc.load_gather(
            tmp, [(li + jnp.int32(16)) * jnp.int32(256) + hb3])
        cs_lo = plsc.cumsum(cw_lo)
        cs_hi = plsc.cumsum(cw_hi) + _scal(cs_lo)
        my_cw = jnp.where(wid < 16, _at(cw_lo, wid), _at(cw_hi, wid - 16))
        my_incl = jnp.where(wid < 16, _at(cs_lo, wid), _at(cs_hi, wid - 16))
        before_w = my_incl - my_cw
        budget = jnp.clip(n_keep_ties - before_w, 0, my_cw)

        zf = jnp.zeros((L,), jnp.float32)
        bouts = [bout0, bout1]
        wsems = [ws0, ws1]

        def stream_simple(strict):
            def go():
                wh = [None, None]

                def process(buf, c):
                    b = c % 2
                    if wh[b] is not None:
                        wh[b].wait()

                    def inner(j, _):
                        vs = [buf[pl.ds(j * (L * _W4) + m * L, L)]
                              for m in range(_W4)]
                        kus = [_ku16(v) for v in vs]
                        for m in range(_W4):
                            keep = kus[m] > t if strict else kus[m] >= t
                            bouts[b][pl.ds(j * (L * _W4) + m * L, L)] = (
                                jnp.where(keep, vs[m], zf))
                        return 0

                    lax.fori_loop(0, _CHUNK // (L * _W4), inner, 0, unroll=2)
                    r, cc = _chunk_rc(wid, c)
                    wh[b] = pltpu.async_copy(
                        bouts[b], y_hbm.at[r, pl.ds(cc, _CHUNK)], wsems[b])

                _stream_in(x_hbm, wid, [bin0, bin1], [rs0, rs1], process)
                for b in range(2):
                    if wh[b] is not None:
                        wh[b].wait()

            return go

        def stream_partial():
            one_i = jnp.ones((L,), jnp.int32)
            zero_i = jnp.zeros((L,), jnp.int32)

            def outer(c, r):
                rr_, cc_ = _chunk_rc(wid, c)
                pltpu.sync_copy(x_hbm.at[rr_, pl.ds(cc_, _CHUNK)], bin0)

                def inner(j, rr):
                    v = bin0[pl.ds(j * L, L)]
                    ku = _ku16(v)
                    tie = ku == t
                    cs = plsc.cumsum(jnp.where(tie, one_i, zero_i))
                    keep = (ku > t) | (tie & ((rr + cs) <= budget))
                    bout0[pl.ds(j * L, L)] = jnp.where(keep, v, zf)
                    return rr + _scal(cs)

                r = lax.fori_loop(0, _CHUNK // L, inner, r)
                pltpu.sync_copy(bout0, y_hbm.at[rr_, pl.ds(cc_, _CHUNK)])
                return r

            lax.fori_loop(0, _NCHUNKS, outer, jnp.int32(0))

        full = budget == my_cw
        none_ = jnp.logical_and(jnp.logical_not(full), budget == 0)
        part = jnp.logical_and(jnp.logical_not(full), budget > 0)

        pl.when(full)(stream_simple(False))
        pl.when(none_)(stream_simple(True))
        pl.when(part)(stream_partial)

    return pass1, pass2, pass3, pass4


@jax.jit
def kernel(hidden_preactivation_BH):
    pass1, pass2, pass3, pass4 = _build_passes()
    x = hidden_preactivation_BH
    h1 = pass1(x)
    h2, sel1 = pass2(x, h1)
    h3, sel2 = pass3(x, h2, sel1)
    return pass4(x, h3, sel2)


# speedup vs baseline: 3.5716x; 1.1375x over previous
import functools

import jax
import jax.numpy as jnp
from jax import lax
from jax.experimental import pallas as pl
from jax.experimental.pallas import tpu as pltpu
from jax.experimental.pallas import tpu_sc as plsc

NC = 2
NS = 16
NW = NC * NS
L = 16

_B = 128
_H = 32768
_N = _B * _H
_PER_W = _N // NW
_CHUNK = 16384
_NCHUNKS = _PER_W // _CHUNK
_ROWS_PER_CHUNK = 1
_KK = 64 * _B
_W4 = 8


def _wid():
    return lax.axis_index("s") * NC + lax.axis_index("c")


def _lanes():
    return lax.iota(jnp.int32, L)


def _ku16(v):
    i = lax.bitcast_convert_type(v, jnp.int32)
    k = i ^ ((i >> 31) & jnp.int32(0x7FFFFFFF))
    return lax.bitcast_convert_type(k, jnp.uint32) ^ jnp.uint32(0x80000000)


def _chunk_rc(wid, c):
    return wid * (_PER_W // _H) + c // (_H // _CHUNK), (c % (_H // _CHUNK)) * _CHUNK


def _zero(ref, nwords):
    z = jnp.zeros((L,), jnp.int32)

    def b(i, _):
        ref[pl.ds(i * L, L)] = z
        return 0

    lax.fori_loop(0, nwords // L, b, 0, unroll=4)


def _scal(v):
    return jnp.max(v)


def _at(v, lane):
    return jnp.sum(jnp.where(_lanes() == lane, v, jnp.zeros_like(v)))


def _pick(v, kkt, running):
    r = lax.rev(v, (0,))
    cs = plsc.cumsum(r)
    m = (running + cs) >= kkt
    lb = _scal(plsc.all_reduce_ffs(m))
    above = running + _at(cs, lb) - _at(r, lb)
    return jnp.int32(15) - lb, above


def _desc_select(mg, gsum, nbuckets, kkt):
    li = _lanes()
    if nbuckets == 4096:
        def bg(g, _):
            acc = jnp.zeros((L,), jnp.int32)
            for l in range(L):
                acc = acc + plsc.load_gather(mg, [(g * L + li) * L + l])
            gsum[pl.ds(g * L, L)] = acc
            return 0

        lax.fori_loop(0, 16, bg, 0)
        ss = jnp.zeros((L,), jnp.int32)
        for l in range(L):
            ss = ss + plsc.load_gather(gsum, [li * L + l])
        s_star, ab0 = _pick(ss, kkt, jnp.int32(0))
        gvec = gsum[pl.ds(s_star * L, L)]
        g_in, ab1 = _pick(gvec, kkt, ab0)
        g_star = s_star * L + g_in
        bvec = mg[pl.ds(g_star * L, L)]
        b_in, ab2 = _pick(bvec, kkt, ab1)
        return g_star * L + b_in, ab2
    else:
        ss = jnp.zeros((L,), jnp.int32)
        for l in range(L):
            ss = ss + plsc.load_gather(mg, [li * L + l])
        g_star, ab0 = _pick(ss, kkt, jnp.int32(0))
        bvec = mg[pl.ds(g_star * L, L)]
        b_in, ab1 = _pick(bvec, kkt, ab0)
        return g_star * L + b_in, ab1


def _global_merge(h_hbm, tmp, mg):
    _zero(mg, 4096)
    for cc in range(8):
        pltpu.sync_copy(h_hbm.at[pl.ds(cc * 16384, 16384)], tmp)

        def b(g, _):
            acc = mg[pl.ds(g * L, L)]
            for w in range(4):
                acc = acc + tmp[pl.ds(w * 4096 + g * L, L)]
            mg[pl.ds(g * L, L)] = acc
            return 0

        lax.fori_loop(0, 256, b, 0, unroll=4)


def _sc_merge_publish(merged, subhist, shm, hsc_hbm):
    sid = lax.axis_index("s")
    cid = lax.axis_index("c")
    pltpu.sync_copy(merged, shm.at[pl.ds(sid * 4096, 4096)])
    plsc.subcore_barrier()
    for r in range(NS):
        pltpu.sync_copy(shm.at[pl.ds(r * 4096 + sid * 256, 256)],
                        subhist.at[pl.ds(r * 256, 256)])

    def rb(g, _):
        acc = jnp.zeros((L,), jnp.int32)
        for r in range(NS):
            acc = acc + subhist[pl.ds(r * 256 + g * L, L)]
        merged[pl.ds(g * L, L)] = acc
        return 0

    lax.fori_loop(0, 16, rb, 0)
    pltpu.sync_copy(merged.at[pl.ds(0, 256)],
                    hsc_hbm.at[pl.ds(cid * 4096 + sid * 256, 256)])


def _global_merge2(hsc_hbm, tmp, mg):
    pltpu.sync_copy(hsc_hbm, tmp.at[pl.ds(0, 2 * 4096)])

    def b(g, _):
        mg[pl.ds(g * L, L)] = (tmp[pl.ds(g * L, L)]
                               + tmp[pl.ds(4096 + g * L, L)])
        return 0

    lax.fori_loop(0, 256, b, 0, unroll=4)


def _lane_merge(subhist, out_ref, nbuckets):
    def b(g, _):
        acc = jnp.zeros((L,), jnp.int32)
        for l in range(L):
            acc = acc + subhist[pl.ds(l * nbuckets + g * L, L)]
        out_ref[pl.ds(g * L, L)] = acc
        return 0

    lax.fori_loop(0, nbuckets // L, b, 0)


def _stream_in(x_hbm, wid, bufs, sems, process):
    r0, c0 = _chunk_rc(wid, 0)
    h = [None, None]
    h[0] = pltpu.async_copy(x_hbm.at[r0, pl.ds(c0, _CHUNK)], bufs[0], sems[0])
    for c in range(_NCHUNKS):
        b = c % 2
        h[b].wait()
        if c + 1 < _NCHUNKS:
            nb = (c + 1) % 2
            rn, cn = _chunk_rc(wid, c + 1)
            h[nb] = pltpu.async_copy(
                x_hbm.at[rn, pl.ds(cn, _CHUNK)], bufs[nb], sems[nb])
        process(bufs[b], c)


def _hist_stream(x_hbm, bufs, sems, subhist, wid, bucket_and_mask):
    ones = jnp.ones((L,), jnp.int32)
    li = _lanes()

    def process(buf, c):
        def inner(j, _):
            vs = [buf[pl.ds(j * (L * _W4) + m * L, L)] for m in range(_W4)]
            kus = [_ku16(v) for v in vs]
            bmns = [bucket_and_mask(ku) for ku in kus]
            for bkt, msk, nb in bmns:
                idx = li * jnp.int32(nb) + bkt
                if msk is None:
                    plsc.addupdate_scatter(subhist, [idx], ones)
                else:
                    plsc.addupdate_scatter(subhist, [idx], ones, mask=msk)
            return 0

        lax.fori_loop(0, _CHUNK // (L * _W4), inner, 0, unroll=2)

    _stream_in(x_hbm, wid, bufs, sems, process)


@functools.cache
def _build_passes():
    mesh = plsc.VectorSubcoreMesh(core_axis_name="c", subcore_axis_name="s")
    cp = pltpu.CompilerParams(needs_layout_passes=False)

    @functools.partial(
        pl.kernel,
        out_type=jax.ShapeDtypeStruct((NC * 4096,), jnp.int32),
        mesh=mesh,
        compiler_params=cp,
        scratch_types=[
            pltpu.VMEM((_CHUNK,), jnp.float32),
            pltpu.VMEM((_CHUNK,), jnp.float32),
            pltpu.VMEM((L * 4096,), jnp.int32),
            pltpu.VMEM((4096,), jnp.int32),
            pltpu.VMEM_SHARED((NS * 4096,), jnp.int32),
            pltpu.SemaphoreType.DMA,
            pltpu.SemaphoreType.DMA,
        ],
    )
    def pass1(x_hbm, h1_hbm, buf0, buf1, subhist, merged, shm, sem0, sem1):
        wid = _wid()
        _zero(subhist, L * 4096)

        def bm(ku):
            b = lax.convert_element_type(ku >> jnp.uint32(20), jnp.int32)
            return b, None, 4096

        _hist_stream(x_hbm, [buf0, buf1], [sem0, sem1], subhist, wid, bm)
        _lane_merge(subhist, merged, 4096)
        _sc_merge_publish(merged, subhist, shm, h1_hbm)

    @functools.partial(
        pl.kernel,
        out_type=[jax.ShapeDtypeStruct((NC * 4096,), jnp.int32),
                  jax.ShapeDtypeStruct((L,), jnp.int32)],
        mesh=mesh,
        compiler_params=cp,
        scratch_types=[
            pltpu.VMEM((_CHUNK,), jnp.float32),
            pltpu.VMEM((_CHUNK,), jnp.float32),
            pltpu.VMEM((16384,), jnp.int32),
            pltpu.VMEM((L * 4096,), jnp.int32),
            pltpu.VMEM((4096,), jnp.int32),
            pltpu.VMEM((256,), jnp.int32),
            pltpu.VMEM((L,), jnp.int32),
            pltpu.VMEM_SHARED((NS * 4096,), jnp.int32),
            pltpu.SemaphoreType.DMA,
            pltpu.SemaphoreType.DMA,
        ],
    )
    def pass2(x_hbm, h1_hbm, h2_hbm, sel1_hbm, buf0, buf1, tmp, subhist, mg,
              gsum, selbuf, shm, sem0, sem1):
        wid = _wid()
        li = _lanes()
        _global_merge2(h1_hbm, tmp, mg)
        hb1, ab1 = _desc_select(mg, gsum, 4096, jnp.int32(_KK))
        hb1u = lax.convert_element_type(hb1, jnp.uint32)

        @pl.when(wid == 0)
        def _():
            z = jnp.zeros((L,), jnp.int32)
            sv = jnp.where(li == 0, hb1, jnp.where(li == 1, ab1, z))
            selbuf[pl.ds(0, L)] = sv
            pltpu.sync_copy(selbuf, sel1_hbm)
        _zero(subhist, L * 4096)

        def bm(ku):
            sel = (ku >> jnp.uint32(20)) == hb1u
            b = lax.convert_element_type(
                (ku >> jnp.uint32(8)) & jnp.uint32(0xFFF), jnp.int32)
            return b, sel, 4096

        _hist_stream(x_hbm, [buf0, buf1], [sem0, sem1], subhist, wid, bm)
        _lane_merge(subhist, mg, 4096)
        _sc_merge_publish(mg, subhist, shm, h2_hbm)

    @functools.partial(
        pl.kernel,
        out_type=[jax.ShapeDtypeStruct((NW * 256,), jnp.int32),
                  jax.ShapeDtypeStruct((L,), jnp.int32)],
        mesh=mesh,
        compiler_params=cp,
        scratch_types=[
            pltpu.VMEM((_CHUNK,), jnp.float32),
            pltpu.VMEM((_CHUNK,), jnp.float32),
            pltpu.VMEM((16384,), jnp.int32),
            pltpu.VMEM((L * 256,), jnp.int32),
            pltpu.VMEM((4096,), jnp.int32),
            pltpu.VMEM((256,), jnp.int32),
            pltpu.VMEM((L,), jnp.int32),
            pltpu.SemaphoreType.DMA,
            pltpu.SemaphoreType.DMA,
        ],
    )
    def pass3(x_hbm, h2_hbm, sel1_hbm, h3_hbm, sel2_hbm, buf0, buf1, tmp,
              subhist, mg, gsum, selbuf, sem0, sem1):
        wid = _wid()
        li = _lanes()
        pltpu.sync_copy(sel1_hbm, selbuf)
        sv1 = selbuf[pl.ds(0, L)]
        hb1 = _at(sv1, 0)
        ab1 = _at(sv1, 1)
        _global_merge2(h2_hbm, tmp, mg)
        hb2, ab2 = _desc_select(mg, gsum, 4096, jnp.int32(_KK) - ab1)
        pref = lax.convert_element_type(hb1 * 4096 + hb2, jnp.uint32)

        @pl.when(wid == 0)
        def _():
            z = jnp.zeros((L,), jnp.int32)
            sv = jnp.where(li == 0, hb1,
                           jnp.where(li == 1, ab1,
                                     jnp.where(li == 2, hb2,
                                               jnp.where(li == 3, ab2, z))))
            selbuf[pl.ds(0, L)] = sv
            pltpu.sync_copy(selbuf, sel2_hbm)
        _zero(subhist, L * 256)

        def bm(ku):
            sel = (ku >> jnp.uint32(8)) == pref
            b = lax.convert_element_type(ku & jnp.uint32(0xFF), jnp.int32)
            return b, sel, 256

        _hist_stream(x_hbm, [buf0, buf1], [sem0, sem1], subhist, wid, bm)
        _lane_merge(subhist, mg, 256)
        pltpu.sync_copy(mg.at[pl.ds(0, 256)], h3_hbm.at[pl.ds(wid * 256, 256)])

    @functools.partial(
        pl.kernel,
        out_type=jax.ShapeDtypeStruct((_B, _H), jnp.float32),
        mesh=mesh,
        compiler_params=cp,
        scratch_types=[
            pltpu.VMEM((_CHUNK,), jnp.float32),
            pltpu.VMEM((_CHUNK,), jnp.float32),
            pltpu.VMEM((_CHUNK,), jnp.float32),
            pltpu.VMEM((_CHUNK,), jnp.float32),
            pltpu.VMEM((16384,), jnp.int32),
            pltpu.VMEM((4096,), jnp.int32),
            pltpu.VMEM((256,), jnp.int32),
            pltpu.VMEM((256,), jnp.int32),
            pltpu.SemaphoreType.DMA,
            pltpu.SemaphoreType.DMA,
            pltpu.SemaphoreType.DMA,
            pltpu.SemaphoreType.DMA,
        ],
    )
    def pass4(x_hbm, h3_hbm, sel2_hbm, y_hbm, bin0, bin1, bout0, bout1,
              tmp, mg, m3, gsum, rs0, rs1, ws0, ws1):
        wid = _wid()
        li = _lanes()

        pltpu.sync_copy(sel2_hbm, mg.at[pl.ds(0, L)])
        sv2 = mg[pl.ds(0, L)]
        hb1 = _at(sv2, 0)
        ab1 = _at(sv2, 1)
        hb2 = _at(sv2, 2)
        ab2 = _at(sv2, 3)

        pltpu.sync_copy(h3_hbm, tmp.at[pl.ds(0, NW * 256)])

        def b3(g, _):
            acc = jnp.zeros((L,), jnp.int32)
            for w in range(NW):
                acc = acc + tmp[pl.ds(w * 256 + g * L, L)]
            m3[pl.ds(g * L, L)] = acc
            return 0

        lax.fori_loop(0, 256 // L, b3, 0)
        kkt3 = jnp.int32(_KK) - ab1 - ab2
        hb3, ab3 = _desc_select(m3, gsum, 256, kkt3)

        t = ((lax.convert_element_type(hb1, jnp.uint32) << jnp.uint32(20))
             | (lax.convert_element_type(hb2, jnp.uint32) << jnp.uint32(8))
             | lax.convert_element_type(hb3, jnp.uint32))
        n_keep_ties = kkt3 - ab3

        cw_lo = plsc.load_gather(tmp, [li * jnp.int32(256) + hb3])
        cw_hi = plsc.load_gather(
            tmp, [(li + jnp.int32(16)) * jnp.int32(256) + hb3])
        cs_lo = plsc.cumsum(cw_lo)
        cs_hi = plsc.cumsum(cw_hi) + _scal(cs_lo)
        my_cw = jnp.where(wid < 16, _at(cw_lo, wid), _at(cw_hi, wid - 16))
        my_incl = jnp.where(wid < 16, _at(cs_lo, wid), _at(cs_hi, wid - 16))
        before_w = my_incl - my_cw
        budget = jnp.clip(n_keep_ties - before_w, 0, my_cw)

        zf = jnp.zeros((L,), jnp.float32)
        bouts = [bout0, bout1]
        wsems = [ws0, ws1]

        def stream_simple(strict):
            def go():
                wh = [None, None]

                def process(buf, c):
                    b = c % 2
                    if wh[b] is not None:
                        wh[b].wait()

                    def inner(j, _):
                        vs = [buf[pl.ds(j * (L * _W4) + m * L, L)]
                              for m in range(_W4)]
                        kus = [_ku16(v) for v in vs]
                        for m in range(_W4):
                            keep = kus[m] > t if strict else kus[m] >= t
                            bouts[b][pl.ds(j * (L * _W4) + m * L, L)] = (
                                jnp.where(keep, vs[m], zf))
                        return 0

                    lax.fori_loop(0, _CHUNK // (L * _W4), inner, 0, unroll=2)
                    r, cc = _chunk_rc(wid, c)
                    wh[b] = pltpu.async_copy(
                        bouts[b], y_hbm.at[r, pl.ds(cc, _CHUNK)], wsems[b])

                _stream_in(x_hbm, wid, [bin0, bin1], [rs0, rs1], process)
                for b in range(2):
                    if wh[b] is not None:
                        wh[b].wait()

            return go

        def stream_partial():
            one_i = jnp.ones((L,), jnp.int32)
            zero_i = jnp.zeros((L,), jnp.int32)

            def outer(c, r):
                rr_, cc_ = _chunk_rc(wid, c)
                pltpu.sync_copy(x_hbm.at[rr_, pl.ds(cc_, _CHUNK)], bin0)

                def inner(j, rr):
                    v = bin0[pl.ds(j * L, L)]
                    ku = _ku16(v)
                    tie = ku == t
                    cs = plsc.cumsum(jnp.where(tie, one_i, zero_i))
                    keep = (ku > t) | (tie & ((rr + cs) <= budget))
                    bout0[pl.ds(j * L, L)] = jnp.where(keep, v, zf)
                    return rr + _scal(cs)

                r = lax.fori_loop(0, _CHUNK // L, inner, r)
                pltpu.sync_copy(bout0, y_hbm.at[rr_, pl.ds(cc_, _CHUNK)])
                return r

            lax.fori_loop(0, _NCHUNKS, outer, jnp.int32(0))

        full = budget == my_cw
        none_ = jnp.logical_and(jnp.logical_not(full), budget == 0)
        part = jnp.logical_and(jnp.logical_not(full), budget > 0)

        pl.when(full)(stream_simple(False))
        pl.when(none_)(stream_simple(True))
        pl.when(part)(stream_partial)

    return pass1, pass2, pass3, pass4


@jax.jit
def kernel(hidden_preactivation_BH):
    pass1, pass2, pass3, pass4 = _build_passes()
    x = hidden_preactivation_BH
    h1 = pass1(x)
    h2, sel1 = pass2(x, h1)
    h3, sel2 = pass3(x, h2, sel1)
    return pass4(x, h3, sel2)


# speedup vs baseline: 3.6941x; 1.0343x over previous
import functools

import jax
import jax.numpy as jnp
from jax import lax
from jax.experimental import pallas as pl
from jax.experimental.pallas import tpu as pltpu
from jax.experimental.pallas import tpu_sc as plsc

NC = 2
NS = 16
NW = NC * NS
L = 16

_B = 128
_H = 32768
_N = _B * _H
_PER_W = _N // NW
_CHUNK = 16384
_NCHUNKS = _PER_W // _CHUNK
_ROWS_PER_CHUNK = 1
_KK = 64 * _B
_W4 = 16


def _wid():
    return lax.axis_index("s") * NC + lax.axis_index("c")


def _lanes():
    return lax.iota(jnp.int32, L)


def _ku16(v):
    i = lax.bitcast_convert_type(v, jnp.int32)
    k = i ^ ((i >> 31) & jnp.int32(0x7FFFFFFF))
    return lax.bitcast_convert_type(k, jnp.uint32) ^ jnp.uint32(0x80000000)


def _chunk_rc(wid, c):
    return wid * (_PER_W // _H) + c // (_H // _CHUNK), (c % (_H // _CHUNK)) * _CHUNK


def _zero(ref, nwords):
    z = jnp.zeros((L,), jnp.int32)

    def b(i, _):
        ref[pl.ds(i * L, L)] = z
        return 0

    lax.fori_loop(0, nwords // L, b, 0, unroll=4)


def _scal(v):
    return jnp.max(v)


def _at(v, lane):
    return jnp.sum(jnp.where(_lanes() == lane, v, jnp.zeros_like(v)))


def _pick(v, kkt, running):
    r = lax.rev(v, (0,))
    cs = plsc.cumsum(r)
    m = (running + cs) >= kkt
    lb = _scal(plsc.all_reduce_ffs(m))
    above = running + _at(cs, lb) - _at(r, lb)
    return jnp.int32(15) - lb, above


def _desc_select(mg, gsum, nbuckets, kkt):
    li = _lanes()
    if nbuckets == 4096:
        def bg(g, _):
            acc = jnp.zeros((L,), jnp.int32)
            for l in range(L):
                acc = acc + plsc.load_gather(mg, [(g * L + li) * L + l])
            gsum[pl.ds(g * L, L)] = acc
            return 0

        lax.fori_loop(0, 16, bg, 0)
        ss = jnp.zeros((L,), jnp.int32)
        for l in range(L):
            ss = ss + plsc.load_gather(gsum, [li * L + l])
        s_star, ab0 = _pick(ss, kkt, jnp.int32(0))
        gvec = gsum[pl.ds(s_star * L, L)]
        g_in, ab1 = _pick(gvec, kkt, ab0)
        g_star = s_star * L + g_in
        bvec = mg[pl.ds(g_star * L, L)]
        b_in, ab2 = _pick(bvec, kkt, ab1)
        return g_star * L + b_in, ab2
    else:
        ss = jnp.zeros((L,), jnp.int32)
        for l in range(L):
            ss = ss + plsc.load_gather(mg, [li * L + l])
        g_star, ab0 = _pick(ss, kkt, jnp.int32(0))
        bvec = mg[pl.ds(g_star * L, L)]
        b_in, ab1 = _pick(bvec, kkt, ab0)
        return g_star * L + b_in, ab1


def _global_merge(h_hbm, tmp, mg):
    _zero(mg, 4096)
    for cc in range(8):
        pltpu.sync_copy(h_hbm.at[pl.ds(cc * 16384, 16384)], tmp)

        def b(g, _):
            acc = mg[pl.ds(g * L, L)]
            for w in range(4):
                acc = acc + tmp[pl.ds(w * 4096 + g * L, L)]
            mg[pl.ds(g * L, L)] = acc
            return 0

        lax.fori_loop(0, 256, b, 0, unroll=4)


def _sc_merge_publish(merged, subhist, shm, hsc_hbm):
    sid = lax.axis_index("s")
    cid = lax.axis_index("c")
    pltpu.sync_copy(merged, shm.at[pl.ds(sid * 4096, 4096)])
    plsc.subcore_barrier()
    for r in range(NS):
        pltpu.sync_copy(shm.at[pl.ds(r * 4096 + sid * 256, 256)],
                        subhist.at[pl.ds(r * 256, 256)])

    def rb(g, _):
        acc = jnp.zeros((L,), jnp.int32)
        for r in range(NS):
            acc = acc + subhist[pl.ds(r * 256 + g * L, L)]
        merged[pl.ds(g * L, L)] = acc
        return 0

    lax.fori_loop(0, 16, rb, 0)
    pltpu.sync_copy(merged.at[pl.ds(0, 256)],
                    hsc_hbm.at[pl.ds(cid * 4096 + sid * 256, 256)])


def _global_merge2(hsc_hbm, tmp, mg):
    pltpu.sync_copy(hsc_hbm, tmp.at[pl.ds(0, 2 * 4096)])

    def b(g, _):
        mg[pl.ds(g * L, L)] = (tmp[pl.ds(g * L, L)]
                               + tmp[pl.ds(4096 + g * L, L)])
        return 0

    lax.fori_loop(0, 256, b, 0, unroll=4)


def _lane_merge(subhist, out_ref, nbuckets):
    def b(g, _):
        acc = jnp.zeros((L,), jnp.int32)
        for l in range(L):
            acc = acc + subhist[pl.ds(l * nbuckets + g * L, L)]
        out_ref[pl.ds(g * L, L)] = acc
        return 0

    lax.fori_loop(0, nbuckets // L, b, 0)


def _stream_in(x_hbm, wid, bufs, sems, process):
    r0, c0 = _chunk_rc(wid, 0)
    h = [None, None]
    h[0] = pltpu.async_copy(x_hbm.at[r0, pl.ds(c0, _CHUNK)], bufs[0], sems[0])
    for c in range(_NCHUNKS):
        b = c % 2
        h[b].wait()
        if c + 1 < _NCHUNKS:
            nb = (c + 1) % 2
            rn, cn = _chunk_rc(wid, c + 1)
            h[nb] = pltpu.async_copy(
                x_hbm.at[rn, pl.ds(cn, _CHUNK)], bufs[nb], sems[nb])
        process(bufs[b], c)


def _hist_stream(x_hbm, bufs, sems, subhist, wid, bucket_and_mask):
    ones = jnp.ones((L,), jnp.int32)
    li = _lanes()

    def process(buf, c):
        def inner(j, _):
            vs = [buf[pl.ds(j * (L * _W4) + m * L, L)] for m in range(_W4)]
            kus = [_ku16(v) for v in vs]
            bmns = [bucket_and_mask(ku) for ku in kus]
            for bkt, msk, nb in bmns:
                idx = li * jnp.int32(nb) + bkt
                if msk is None:
                    plsc.addupdate_scatter(subhist, [idx], ones)
                else:
                    plsc.addupdate_scatter(subhist, [idx], ones, mask=msk)
            return 0

        lax.fori_loop(0, _CHUNK // (L * _W4), inner, 0, unroll=2)

    _stream_in(x_hbm, wid, bufs, sems, process)


@functools.cache
def _build_passes():
    mesh = plsc.VectorSubcoreMesh(core_axis_name="c", subcore_axis_name="s")
    cp = pltpu.CompilerParams(needs_layout_passes=False)

    @functools.partial(
        pl.kernel,
        out_type=jax.ShapeDtypeStruct((NC * 4096,), jnp.int32),
        mesh=mesh,
        compiler_params=cp,
        scratch_types=[
            pltpu.VMEM((_CHUNK,), jnp.float32),
            pltpu.VMEM((_CHUNK,), jnp.float32),
            pltpu.VMEM((L * 4096,), jnp.int32),
            pltpu.VMEM((4096,), jnp.int32),
            pltpu.VMEM_SHARED((NS * 4096,), jnp.int32),
            pltpu.SemaphoreType.DMA,
            pltpu.SemaphoreType.DMA,
        ],
    )
    def pass1(x_hbm, h1_hbm, buf0, buf1, subhist, merged, shm, sem0, sem1):
        wid = _wid()
        _zero(subhist, L * 4096)

        def bm(ku):
            b = lax.convert_element_type(ku >> jnp.uint32(20), jnp.int32)
            return b, None, 4096

        _hist_stream(x_hbm, [buf0, buf1], [sem0, sem1], subhist, wid, bm)
        _lane_merge(subhist, merged, 4096)
        _sc_merge_publish(merged, subhist, shm, h1_hbm)

    @functools.partial(
        pl.kernel,
        out_type=[jax.ShapeDtypeStruct((NC * 4096,), jnp.int32),
                  jax.ShapeDtypeStruct((L,), jnp.int32)],
        mesh=mesh,
        compiler_params=cp,
        scratch_types=[
            pltpu.VMEM((_CHUNK,), jnp.float32),
            pltpu.VMEM((_CHUNK,), jnp.float32),
            pltpu.VMEM((16384,), jnp.int32),
            pltpu.VMEM((L * 4096,), jnp.int32),
            pltpu.VMEM((4096,), jnp.int32),
            pltpu.VMEM((256,), jnp.int32),
            pltpu.VMEM((L,), jnp.int32),
            pltpu.VMEM_SHARED((NS * 4096,), jnp.int32),
            pltpu.SemaphoreType.DMA,
            pltpu.SemaphoreType.DMA,
        ],
    )
    def pass2(x_hbm, h1_hbm, h2_hbm, sel1_hbm, buf0, buf1, tmp, subhist, mg,
              gsum, selbuf, shm, sem0, sem1):
        wid = _wid()
        li = _lanes()
        _global_merge2(h1_hbm, tmp, mg)
        hb1, ab1 = _desc_select(mg, gsum, 4096, jnp.int32(_KK))
        hb1u = lax.convert_element_type(hb1, jnp.uint32)

        @pl.when(wid == 0)
        def _():
            z = jnp.zeros((L,), jnp.int32)
            sv = jnp.where(li == 0, hb1, jnp.where(li == 1, ab1, z))
            selbuf[pl.ds(0, L)] = sv
            pltpu.sync_copy(selbuf, sel1_hbm)
        _zero(subhist, L * 4096)

        def bm(ku):
            sel = (ku >> jnp.uint32(20)) == hb1u
            b = lax.convert_element_type(
                (ku >> jnp.uint32(8)) & jnp.uint32(0xFFF), jnp.int32)
            return b, sel, 4096

        _hist_stream(x_hbm, [buf0, buf1], [sem0, sem1], subhist, wid, bm)
        _lane_merge(subhist, mg, 4096)
        _sc_merge_publish(mg, subhist, shm, h2_hbm)

    @functools.partial(
        pl.kernel,
        out_type=[jax.ShapeDtypeStruct((NW * 256,), jnp.int32),
                  jax.ShapeDtypeStruct((L,), jnp.int32)],
        mesh=mesh,
        compiler_params=cp,
        scratch_types=[
            pltpu.VMEM((_CHUNK,), jnp.float32),
            pltpu.VMEM((_CHUNK,), jnp.float32),
            pltpu.VMEM((16384,), jnp.int32),
            pltpu.VMEM((L * 256,), jnp.int32),
            pltpu.VMEM((4096,), jnp.int32),
            pltpu.VMEM((256,), jnp.int32),
            pltpu.VMEM((L,), jnp.int32),
            pltpu.SemaphoreType.DMA,
            pltpu.SemaphoreType.DMA,
        ],
    )
    def pass3(x_hbm, h2_hbm, sel1_hbm, h3_hbm, sel2_hbm, buf0, buf1, tmp,
              subhist, mg, gsum, selbuf, sem0, sem1):
        wid = _wid()
        li = _lanes()
        pltpu.sync_copy(sel1_hbm, selbuf)
        sv1 = selbuf[pl.ds(0, L)]
        hb1 = _at(sv1, 0)
        ab1 = _at(sv1, 1)
        _global_merge2(h2_hbm, tmp, mg)
        hb2, ab2 = _desc_select(mg, gsum, 4096, jnp.int32(_KK) - ab1)
        pref = lax.convert_element_type(hb1 * 4096 + hb2, jnp.uint32)

        @pl.when(wid == 0)
        def _():
            z = jnp.zeros((L,), jnp.int32)
            sv = jnp.where(li == 0, hb1,
                           jnp.where(li == 1, ab1,
                                     jnp.where(li == 2, hb2,
                                               jnp.where(li == 3, ab2, z))))
            selbuf[pl.ds(0, L)] = sv
            pltpu.sync_copy(selbuf, sel2_hbm)
        _zero(subhist, L * 256)

        def bm(ku):
            sel = (ku >> jnp.uint32(8)) == pref
            b = lax.convert_element_type(ku & jnp.uint32(0xFF), jnp.int32)
            return b, sel, 256

        _hist_stream(x_hbm, [buf0, buf1], [sem0, sem1], subhist, wid, bm)
        _lane_merge(subhist, mg, 256)
        pltpu.sync_copy(mg.at[pl.ds(0, 256)], h3_hbm.at[pl.ds(wid * 256, 256)])

    @functools.partial(
        pl.kernel,
        out_type=jax.ShapeDtypeStruct((_B, _H), jnp.float32),
        mesh=mesh,
        compiler_params=cp,
        scratch_types=[
            pltpu.VMEM((_CHUNK,), jnp.float32),
            pltpu.VMEM((_CHUNK,), jnp.float32),
            pltpu.VMEM((_CHUNK,), jnp.float32),
            pltpu.VMEM((_CHUNK,), jnp.float32),
            pltpu.VMEM((16384,), jnp.int32),
            pltpu.VMEM((4096,), jnp.int32),
            pltpu.VMEM((256,), jnp.int32),
            pltpu.VMEM((256,), jnp.int32),
            pltpu.SemaphoreType.DMA,
            pltpu.SemaphoreType.DMA,
            pltpu.SemaphoreType.DMA,
            pltpu.SemaphoreType.DMA,
        ],
    )
    def pass4(x_hbm, h3_hbm, sel2_hbm, y_hbm, bin0, bin1, bout0, bout1,
              tmp, mg, m3, gsum, rs0, rs1, ws0, ws1):
        wid = _wid()
        li = _lanes()

        pltpu.sync_copy(sel2_hbm, mg.at[pl.ds(0, L)])
        sv2 = mg[pl.ds(0, L)]
        hb1 = _at(sv2, 0)
        ab1 = _at(sv2, 1)
        hb2 = _at(sv2, 2)
        ab2 = _at(sv2, 3)

        pltpu.sync_copy(h3_hbm, tmp.at[pl.ds(0, NW * 256)])

        def b3(g, _):
            acc = jnp.zeros((L,), jnp.int32)
            for w in range(NW):
                acc = acc + tmp[pl.ds(w * 256 + g * L, L)]
            m3[pl.ds(g * L, L)] = acc
            return 0

        lax.fori_loop(0, 256 // L, b3, 0)
        kkt3 = jnp.int32(_KK) - ab1 - ab2
        hb3, ab3 = _desc_select(m3, gsum, 256, kkt3)

        t = ((lax.convert_element_type(hb1, jnp.uint32) << jnp.uint32(20))
             | (lax.convert_element_type(hb2, jnp.uint32) << jnp.uint32(8))
             | lax.convert_element_type(hb3, jnp.uint32))
        n_keep_ties = kkt3 - ab3

        cw_lo = plsc.load_gather(tmp, [li * jnp.int32(256) + hb3])
        cw_hi = plsc.load_gather(
            tmp, [(li + jnp.int32(16)) * jnp.int32(256) + hb3])
        cs_lo = plsc.cumsum(cw_lo)
        cs_hi = plsc.cumsum(cw_hi) + _scal(cs_lo)
        my_cw = jnp.where(wid < 16, _at(cw_lo, wid), _at(cw_hi, wid - 16))
        my_incl = jnp.where(wid < 16, _at(cs_lo, wid), _at(cs_hi, wid - 16))
        before_w = my_incl - my_cw
        budget = jnp.clip(n_keep_ties - before_w, 0, my_cw)

        zf = jnp.zeros((L,), jnp.float32)
        bouts = [bout0, bout1]
        wsems = [ws0, ws1]

        def stream_simple(strict):
            def go():
                wh = [None, None]

                def process(buf, c):
                    b = c % 2
                    if wh[b] is not None:
                        wh[b].wait()

                    def inner(j, _):
                        vs = [buf[pl.ds(j * (L * _W4) + m * L, L)]
                              for m in range(_W4)]
                        kus = [_ku16(v) for v in vs]
                        for m in range(_W4):
                            keep = kus[m] > t if strict else kus[m] >= t
                            bouts[b][pl.ds(j * (L * _W4) + m * L, L)] = (
                                jnp.where(keep, vs[m], zf))
                        return 0

                    lax.fori_loop(0, _CHUNK // (L * _W4), inner, 0, unroll=2)
                    r, cc = _chunk_rc(wid, c)
                    wh[b] = pltpu.async_copy(
                        bouts[b], y_hbm.at[r, pl.ds(cc, _CHUNK)], wsems[b])

                _stream_in(x_hbm, wid, [bin0, bin1], [rs0, rs1], process)
                for b in range(2):
                    if wh[b] is not None:
                        wh[b].wait()

            return go

        def stream_partial():
            one_i = jnp.ones((L,), jnp.int32)
            zero_i = jnp.zeros((L,), jnp.int32)

            def outer(c, r):
                rr_, cc_ = _chunk_rc(wid, c)
                pltpu.sync_copy(x_hbm.at[rr_, pl.ds(cc_, _CHUNK)], bin0)

                def inner(j, rr):
                    v = bin0[pl.ds(j * L, L)]
                    ku = _ku16(v)
                    tie = ku == t
                    cs = plsc.cumsum(jnp.where(tie, one_i, zero_i))
                    keep = (ku > t) | (tie & ((rr + cs) <= budget))
                    bout0[pl.ds(j * L, L)] = jnp.where(keep, v, zf)
                    return rr + _scal(cs)

                r = lax.fori_loop(0, _CHUNK // L, inner, r)
                pltpu.sync_copy(bout0, y_hbm.at[rr_, pl.ds(cc_, _CHUNK)])
                return r

            lax.fori_loop(0, _NCHUNKS, outer, jnp.int32(0))

        full = budget == my_cw
        none_ = jnp.logical_and(jnp.logical_not(full), budget == 0)
        part = jnp.logical_and(jnp.logical_not(full), budget > 0)

        pl.when(full)(stream_simple(False))
        pl.when(none_)(stream_simple(True))
        pl.when(part)(stream_partial)

    return pass1, pass2, pass3, pass4


@jax.jit
def kernel(hidden_preactivation_BH):
    pass1, pass2, pass3, pass4 = _build_passes()
    x = hidden_preactivation_BH
    h1 = pass1(x)
    h2, sel1 = pass2(x, h1)
    h3, sel2 = pass3(x, h2, sel1)
    return pass4(x, h3, sel2)


# speedup vs baseline: 4.0242x; 1.0894x over previous
import functools

import jax
import jax.numpy as jnp
from jax import lax
from jax.experimental import pallas as pl
from jax.experimental.pallas import tpu as pltpu
from jax.experimental.pallas import tpu_sc as plsc

NC = 2
NS = 16
NW = NC * NS
L = 16

_B = 128
_H = 32768
_N = _B * _H
_PER_W = _N // NW
_CHUNK = 16384
_NCHUNKS = _PER_W // _CHUNK
_ROWS_PER_CHUNK = 1
_KK = 64 * _B
_W4 = 16
_CAP = 512
_CAPV = _CAP // L


def _wid():
    return lax.axis_index("s") * NC + lax.axis_index("c")


def _lanes():
    return lax.iota(jnp.int32, L)


def _ku16(v):
    i = lax.bitcast_convert_type(v, jnp.int32)
    k = i ^ ((i >> 31) & jnp.int32(0x7FFFFFFF))
    return lax.bitcast_convert_type(k, jnp.uint32) ^ jnp.uint32(0x80000000)


def _chunk_rc(wid, c):
    return wid * (_PER_W // _H) + c // (_H // _CHUNK), (c % (_H // _CHUNK)) * _CHUNK


def _zero(ref, nwords):
    z = jnp.zeros((L,), jnp.int32)

    def b(i, _):
        ref[pl.ds(i * L, L)] = z
        return 0

    lax.fori_loop(0, nwords // L, b, 0, unroll=4)


def _scal(v):
    return jnp.max(v)


def _lane0(v):
    return lax.squeeze(lax.slice(v, (0,), (1,)), (0,))


def _inv_ku16(ku):
    k = lax.bitcast_convert_type(ku ^ jnp.uint32(0x80000000), jnp.int32)
    i = k ^ ((k >> 31) & jnp.int32(0x7FFFFFFF))
    return lax.bitcast_convert_type(i, jnp.float32)


def _at(v, lane):
    return jnp.sum(jnp.where(_lanes() == lane, v, jnp.zeros_like(v)))


def _pick(v, kkt, running):
    r = lax.rev(v, (0,))
    cs = plsc.cumsum(r)
    m = (running + cs) >= kkt
    lb = _scal(plsc.all_reduce_ffs(m))
    above = running + _at(cs, lb) - _at(r, lb)
    return jnp.int32(15) - lb, above


def _desc_select(mg, gsum, nbuckets, kkt):
    li = _lanes()
    if nbuckets == 4096:
        def bg(g, _):
            acc = jnp.zeros((L,), jnp.int32)
            for l in range(L):
                acc = acc + plsc.load_gather(mg, [(g * L + li) * L + l])
            gsum[pl.ds(g * L, L)] = acc
            return 0

        lax.fori_loop(0, 16, bg, 0)
        ss = jnp.zeros((L,), jnp.int32)
        for l in range(L):
            ss = ss + plsc.load_gather(gsum, [li * L + l])
        s_star, ab0 = _pick(ss, kkt, jnp.int32(0))
        gvec = gsum[pl.ds(s_star * L, L)]
        g_in, ab1 = _pick(gvec, kkt, ab0)
        g_star = s_star * L + g_in
        bvec = mg[pl.ds(g_star * L, L)]
        b_in, ab2 = _pick(bvec, kkt, ab1)
        return g_star * L + b_in, ab2
    else:
        ss = jnp.zeros((L,), jnp.int32)
        for l in range(L):
            ss = ss + plsc.load_gather(mg, [li * L + l])
        g_star, ab0 = _pick(ss, kkt, jnp.int32(0))
        bvec = mg[pl.ds(g_star * L, L)]
        b_in, ab1 = _pick(bvec, kkt, ab0)
        return g_star * L + b_in, ab1


def _global_merge(h_hbm, tmp, mg):
    _zero(mg, 4096)
    for cc in range(8):
        pltpu.sync_copy(h_hbm.at[pl.ds(cc * 16384, 16384)], tmp)

        def b(g, _):
            acc = mg[pl.ds(g * L, L)]
            for w in range(4):
                acc = acc + tmp[pl.ds(w * 4096 + g * L, L)]
            mg[pl.ds(g * L, L)] = acc
            return 0

        lax.fori_loop(0, 256, b, 0, unroll=4)


def _sc_merge_publish(merged, subhist, shm, hsc_hbm):
    sid = lax.axis_index("s")
    cid = lax.axis_index("c")
    pltpu.sync_copy(merged, shm.at[pl.ds(sid * 4096, 4096)])
    plsc.subcore_barrier()
    for r in range(NS):
        pltpu.sync_copy(shm.at[pl.ds(r * 4096 + sid * 256, 256)],
                        subhist.at[pl.ds(r * 256, 256)])

    def rb(g, _):
        acc = jnp.zeros((L,), jnp.int32)
        for r in range(NS):
            acc = acc + subhist[pl.ds(r * 256 + g * L, L)]
        merged[pl.ds(g * L, L)] = acc
        return 0

    lax.fori_loop(0, 16, rb, 0)
    pltpu.sync_copy(merged.at[pl.ds(0, 256)],
                    hsc_hbm.at[pl.ds(cid * 4096 + sid * 256, 256)])


def _global_merge2(hsc_hbm, tmp, mg):
    pltpu.sync_copy(hsc_hbm, tmp.at[pl.ds(0, 2 * 4096)])

    def b(g, _):
        mg[pl.ds(g * L, L)] = (tmp[pl.ds(g * L, L)]
                               + tmp[pl.ds(4096 + g * L, L)])
        return 0

    lax.fori_loop(0, 256, b, 0, unroll=4)


def _lane_merge(subhist, out_ref, nbuckets):
    def b(g, _):
        acc = jnp.zeros((L,), jnp.int32)
        for l in range(L):
            acc = acc + subhist[pl.ds(l * nbuckets + g * L, L)]
        out_ref[pl.ds(g * L, L)] = acc
        return 0

    lax.fori_loop(0, nbuckets // L, b, 0)


def _stream_in(x_hbm, wid, bufs, sems, process):
    r0, c0 = _chunk_rc(wid, 0)
    h = [None, None]
    h[0] = pltpu.async_copy(x_hbm.at[r0, pl.ds(c0, _CHUNK)], bufs[0], sems[0])
    for c in range(_NCHUNKS):
        b = c % 2
        h[b].wait()
        if c + 1 < _NCHUNKS:
            nb = (c + 1) % 2
            rn, cn = _chunk_rc(wid, c + 1)
            h[nb] = pltpu.async_copy(
                x_hbm.at[rn, pl.ds(cn, _CHUNK)], bufs[nb], sems[nb])
        process(bufs[b], c)


def _hist_stream(x_hbm, bufs, sems, subhist, wid, bucket_and_mask):
    ones = jnp.ones((L,), jnp.int32)
    li = _lanes()

    def process(buf, c):
        def inner(j, _):
            vs = [buf[pl.ds(j * (L * _W4) + m * L, L)] for m in range(_W4)]
            kus = [_ku16(v) for v in vs]
            bmns = [bucket_and_mask(ku) for ku in kus]
            for bkt, msk, nb in bmns:
                idx = li * jnp.int32(nb) + bkt
                if msk is None:
                    plsc.addupdate_scatter(subhist, [idx], ones)
                else:
                    plsc.addupdate_scatter(subhist, [idx], ones, mask=msk)
            return 0

        lax.fori_loop(0, _CHUNK // (L * _W4), inner, 0, unroll=2)

    _stream_in(x_hbm, wid, bufs, sems, process)


@functools.cache
def _build_passes():
    mesh = plsc.VectorSubcoreMesh(core_axis_name="c", subcore_axis_name="s")
    cp = pltpu.CompilerParams(needs_layout_passes=False)

    @functools.partial(
        pl.kernel,
        out_type=jax.ShapeDtypeStruct((NC * 4096,), jnp.int32),
        mesh=mesh,
        compiler_params=cp,
        scratch_types=[
            pltpu.VMEM((_CHUNK,), jnp.float32),
            pltpu.VMEM((_CHUNK,), jnp.float32),
            pltpu.VMEM((L * 4096,), jnp.int32),
            pltpu.VMEM((4096,), jnp.int32),
            pltpu.VMEM_SHARED((NS * 4096,), jnp.int32),
            pltpu.SemaphoreType.DMA,
            pltpu.SemaphoreType.DMA,
        ],
    )
    def pass1(x_hbm, h1_hbm, buf0, buf1, subhist, merged, shm, sem0, sem1):
        wid = _wid()
        _zero(subhist, L * 4096)

        def bm(ku):
            b = lax.convert_element_type(ku >> jnp.uint32(20), jnp.int32)
            return b, None, 4096

        _hist_stream(x_hbm, [buf0, buf1], [sem0, sem1], subhist, wid, bm)
        _lane_merge(subhist, merged, 4096)
        _sc_merge_publish(merged, subhist, shm, h1_hbm)

    @functools.partial(
        pl.kernel,
        out_type=[jax.ShapeDtypeStruct((NC * 4096,), jnp.int32),
                  jax.ShapeDtypeStruct((L,), jnp.int32),
                  jax.ShapeDtypeStruct((NW * _CAP,), jnp.int32),
                  jax.ShapeDtypeStruct((NW * _CAP,), jnp.int32),
                  jax.ShapeDtypeStruct((NW * L,), jnp.int32)],
        mesh=mesh,
        compiler_params=cp,
        scratch_types=[
            pltpu.VMEM((_CHUNK,), jnp.float32),
            pltpu.VMEM((_CHUNK,), jnp.float32),
            pltpu.VMEM((16384,), jnp.int32),
            pltpu.VMEM((L * 4096,), jnp.int32),
            pltpu.VMEM((4096,), jnp.int32),
            pltpu.VMEM((256,), jnp.int32),
            pltpu.VMEM((L,), jnp.int32),
            pltpu.VMEM((_CAP + L,), jnp.int32),
            pltpu.VMEM((_CAP + L,), jnp.int32),
            pltpu.VMEM_SHARED((NS * 4096,), jnp.int32),
            pltpu.SemaphoreType.DMA,
            pltpu.SemaphoreType.DMA,
        ],
    )
    def pass2(x_hbm, h1_hbm, h2_hbm, sel1_hbm, lv_hbm, lix_hbm, cnt_hbm,
              buf0, buf1, tmp, subhist, mg, gsum, selbuf, lvbuf, lixbuf, shm,
              sem0, sem1):
        wid = _wid()
        li = _lanes()
        _global_merge2(h1_hbm, tmp, mg)
        hb1, ab1 = _desc_select(mg, gsum, 4096, jnp.int32(_KK))
        hb1u = lax.convert_element_type(hb1, jnp.uint32)

        @pl.when(wid == 0)
        def _():
            z = jnp.zeros((L,), jnp.int32)
            sv = jnp.where(li == 0, hb1, jnp.where(li == 1, ab1, z))
            selbuf[pl.ds(0, L)] = sv
            pltpu.sync_copy(selbuf, sel1_hbm)

        _zero(subhist, L * 4096)
        ones = jnp.ones((L,), jnp.int32)

        def process(buf, c):
            def inner(j, cur):
                vs = [buf[pl.ds(j * (L * _W4) + m * L, L)] for m in range(_W4)]
                kus = [_ku16(v) for v in vs]
                for m in range(_W4):
                    ku = kus[m]
                    b12 = ku >> jnp.uint32(20)
                    sel = b12 == hb1u
                    b = lax.convert_element_type(
                        (ku >> jnp.uint32(8)) & jnp.uint32(0xFFF), jnp.int32)
                    idx = li * jnp.int32(4096) + b
                    plsc.addupdate_scatter(subhist, [idx], ones, mask=sel)
                    selge = b12 >= hb1u
                    addr = jnp.minimum(cur, jnp.int32(_CAP))
                    lidx = (jnp.int32(c * _CHUNK + m * L)
                            + j * jnp.int32(L * _W4) + li)
                    kui = lax.bitcast_convert_type(ku, jnp.int32)
                    plsc.store_compressed(lvbuf.at[pl.ds(addr, L)], kui, mask=selge)
                    plsc.store_compressed(lixbuf.at[pl.ds(addr, L)], lidx,
                                          mask=selge)
                    cur = cur + _lane0(
                        plsc.all_reduce_population_count(selge))
                return cur

            return lax.fori_loop(0, _CHUNK // (L * _W4), inner, cur0,
                                 unroll=2)

        bufs = [buf0, buf1]
        sems = [sem0, sem1]
        r0, c0 = _chunk_rc(wid, 0)
        h = [None, None]
        h[0] = pltpu.async_copy(x_hbm.at[r0, pl.ds(c0, _CHUNK)], bufs[0],
                                sems[0])
        ntot = jnp.int32(0)
        for c in range(_NCHUNKS):
            b = c % 2
            h[b].wait()
            if c + 1 < _NCHUNKS:
                nb = (c + 1) % 2
                rn, cn = _chunk_rc(wid, c + 1)
                h[nb] = pltpu.async_copy(
                    x_hbm.at[rn, pl.ds(cn, _CHUNK)], bufs[nb], sems[nb])
            cur0 = ntot
            ntot = process(bufs[b], c)

        _lane_merge(subhist, mg, 4096)
        _sc_merge_publish(mg, subhist, shm, h2_hbm)
        pltpu.sync_copy(lvbuf.at[pl.ds(0, _CAP)],
                        lv_hbm.at[pl.ds(wid * _CAP, _CAP)])
        pltpu.sync_copy(lixbuf.at[pl.ds(0, _CAP)],
                        lix_hbm.at[pl.ds(wid * _CAP, _CAP)])
        cv = jnp.where(li == 0, ntot, jnp.zeros((L,), jnp.int32))
        selbuf[pl.ds(0, L)] = cv
        pltpu.sync_copy(selbuf, cnt_hbm.at[pl.ds(wid * L, L)])

    @functools.partial(
        pl.kernel,
        out_type=[jax.ShapeDtypeStruct((NW * 256,), jnp.int32),
                  jax.ShapeDtypeStruct((L,), jnp.int32)],
        mesh=mesh,
        compiler_params=cp,
        scratch_types=[
            pltpu.VMEM((_CHUNK,), jnp.float32),
            pltpu.VMEM((_CHUNK,), jnp.float32),
            pltpu.VMEM((16384,), jnp.int32),
            pltpu.VMEM((L * 256,), jnp.int32),
            pltpu.VMEM((4096,), jnp.int32),
            pltpu.VMEM((256,), jnp.int32),
            pltpu.VMEM((L,), jnp.int32),
            pltpu.SemaphoreType.DMA,
            pltpu.SemaphoreType.DMA,
        ],
    )
    def pass3(x_hbm, h2_hbm, sel1_hbm, h3_hbm, sel2_hbm, buf0, buf1, tmp,
              subhist, mg, gsum, selbuf, sem0, sem1):
        wid = _wid()
        li = _lanes()
        pltpu.sync_copy(sel1_hbm, selbuf)
        sv1 = selbuf[pl.ds(0, L)]
        hb1 = _at(sv1, 0)
        ab1 = _at(sv1, 1)
        _global_merge2(h2_hbm, tmp, mg)
        hb2, ab2 = _desc_select(mg, gsum, 4096, jnp.int32(_KK) - ab1)
        pref = lax.convert_element_type(hb1 * 4096 + hb2, jnp.uint32)

        @pl.when(wid == 0)
        def _():
            z = jnp.zeros((L,), jnp.int32)
            sv = jnp.where(li == 0, hb1,
                           jnp.where(li == 1, ab1,
                                     jnp.where(li == 2, hb2,
                                               jnp.where(li == 3, ab2, z))))
            selbuf[pl.ds(0, L)] = sv
            pltpu.sync_copy(selbuf, sel2_hbm)
        _zero(subhist, L * 256)

        def bm(ku):
            sel = (ku >> jnp.uint32(8)) == pref
            b = lax.convert_element_type(ku & jnp.uint32(0xFF), jnp.int32)
            return b, sel, 256

        _hist_stream(x_hbm, [buf0, buf1], [sem0, sem1], subhist, wid, bm)
        _lane_merge(subhist, mg, 256)
        pltpu.sync_copy(mg.at[pl.ds(0, 256)], h3_hbm.at[pl.ds(wid * 256, 256)])

    @functools.partial(
        pl.kernel,
        out_type=jax.ShapeDtypeStruct((_B, _H), jnp.float32),
        mesh=mesh,
        compiler_params=cp,
        scratch_types=[
            pltpu.VMEM((_CHUNK,), jnp.float32),
            pltpu.VMEM((_CHUNK,), jnp.float32),
            pltpu.VMEM((_CHUNK,), jnp.float32),
            pltpu.VMEM((_CHUNK,), jnp.float32),
            pltpu.VMEM((16384,), jnp.int32),
            pltpu.VMEM((4096,), jnp.int32),
            pltpu.VMEM((256,), jnp.int32),
            pltpu.VMEM((256,), jnp.int32),
            pltpu.SemaphoreType.DMA,
            pltpu.SemaphoreType.DMA,
            pltpu.SemaphoreType.DMA,
            pltpu.SemaphoreType.DMA,
        ],
    )
    def pass4(x_hbm, h3_hbm, sel2_hbm, y_hbm, bin0, bin1, bout0, bout1,
              tmp, mg, m3, gsum, rs0, rs1, ws0, ws1):
        wid = _wid()
        li = _lanes()

        pltpu.sync_copy(sel2_hbm, mg.at[pl.ds(0, L)])
        sv2 = mg[pl.ds(0, L)]
        hb1 = _at(sv2, 0)
        ab1 = _at(sv2, 1)
        hb2 = _at(sv2, 2)
        ab2 = _at(sv2, 3)

        pltpu.sync_copy(h3_hbm, tmp.at[pl.ds(0, NW * 256)])

        def b3(g, _):
            acc = jnp.zeros((L,), jnp.int32)
            for w in range(NW):
                acc = acc + tmp[pl.ds(w * 256 + g * L, L)]
            m3[pl.ds(g * L, L)] = acc
            return 0

        lax.fori_loop(0, 256 // L, b3, 0)
        kkt3 = jnp.int32(_KK) - ab1 - ab2
        hb3, ab3 = _desc_select(m3, gsum, 256, kkt3)

        t = ((lax.convert_element_type(hb1, jnp.uint32) << jnp.uint32(20))
             | (lax.convert_element_type(hb2, jnp.uint32) << jnp.uint32(8))
             | lax.convert_element_type(hb3, jnp.uint32))
        n_keep_ties = kkt3 - ab3

        cw_lo = plsc.load_gather(tmp, [li * jnp.int32(256) + hb3])
        cw_hi = plsc.load_gather(
            tmp, [(li + jnp.int32(16)) * jnp.int32(256) + hb3])
        cs_lo = plsc.cumsum(cw_lo)
        cs_hi = plsc.cumsum(cw_hi) + _scal(cs_lo)
        my_cw = jnp.where(wid < 16, _at(cw_lo, wid), _at(cw_hi, wid - 16))
        my_incl = jnp.where(wid < 16, _at(cs_lo, wid), _at(cs_hi, wid - 16))
        before_w = my_incl - my_cw
        budget = jnp.clip(n_keep_ties - before_w, 0, my_cw)

        zf = jnp.zeros((L,), jnp.float32)
        bouts = [bout0, bout1]
        wsems = [ws0, ws1]

        def stream_simple(strict):
            def go():
                wh = [None, None]

                def process(buf, c):
                    b = c % 2
                    if wh[b] is not None:
                        wh[b].wait()

                    def inner(j, _):
                        vs = [buf[pl.ds(j * (L * _W4) + m * L, L)]
                              for m in range(_W4)]
                        kus = [_ku16(v) for v in vs]
                        for m in range(_W4):
                            keep = kus[m] > t if strict else kus[m] >= t
                            bouts[b][pl.ds(j * (L * _W4) + m * L, L)] = (
                                jnp.where(keep, vs[m], zf))
                        return 0

                    lax.fori_loop(0, _CHUNK // (L * _W4), inner, 0, unroll=2)
                    r, cc = _chunk_rc(wid, c)
                    wh[b] = pltpu.async_copy(
                        bouts[b], y_hbm.at[r, pl.ds(cc, _CHUNK)], wsems[b])

                _stream_in(x_hbm, wid, [bin0, bin1], [rs0, rs1], process)
                for b in range(2):
                    if wh[b] is not None:
                        wh[b].wait()

            return go

        def stream_partial():
            one_i = jnp.ones((L,), jnp.int32)
            zero_i = jnp.zeros((L,), jnp.int32)

            def outer(c, r):
                rr_, cc_ = _chunk_rc(wid, c)
                pltpu.sync_copy(x_hbm.at[rr_, pl.ds(cc_, _CHUNK)], bin0)

                def inner(j, rr):
                    v = bin0[pl.ds(j * L, L)]
                    ku = _ku16(v)
                    tie = ku == t
                    cs = plsc.cumsum(jnp.where(tie, one_i, zero_i))
                    keep = (ku > t) | (tie & ((rr + cs) <= budget))
                    bout0[pl.ds(j * L, L)] = jnp.where(keep, v, zf)
                    return rr + _scal(cs)

                r = lax.fori_loop(0, _CHUNK // L, inner, r)
                pltpu.sync_copy(bout0, y_hbm.at[rr_, pl.ds(cc_, _CHUNK)])
                return r

            lax.fori_loop(0, _NCHUNKS, outer, jnp.int32(0))

        full = budget == my_cw
        none_ = jnp.logical_and(jnp.logical_not(full), budget == 0)
        part = jnp.logical_and(jnp.logical_not(full), budget > 0)

        pl.when(full)(stream_simple(False))
        pl.when(none_)(stream_simple(True))
        pl.when(part)(stream_partial)


    @functools.partial(
        pl.kernel,
        out_type=[jax.ShapeDtypeStruct((NW * 256,), jnp.int32),
                  jax.ShapeDtypeStruct((L,), jnp.int32)],
        mesh=mesh,
        compiler_params=cp,
        scratch_types=[
            pltpu.VMEM((16384,), jnp.int32),
            pltpu.VMEM((L * 256,), jnp.int32),
            pltpu.VMEM((4096,), jnp.int32),
            pltpu.VMEM((256,), jnp.int32),
            pltpu.VMEM((L,), jnp.int32),
            pltpu.VMEM((_CAP,), jnp.int32),
        ],
    )
    def pass3n(h2_hbm, sel1_hbm, lv_hbm, cnt_hbm, h3_hbm, sel2_hbm,
               tmp, subhist, mg, gsum, selbuf, lbuf):
        wid = _wid()
        li = _lanes()
        pltpu.sync_copy(sel1_hbm, selbuf)
        sv1 = selbuf[pl.ds(0, L)]
        hb1 = _at(sv1, 0)
        ab1 = _at(sv1, 1)
        _global_merge2(h2_hbm, tmp, mg)
        hb2, ab2 = _desc_select(mg, gsum, 4096, jnp.int32(_KK) - ab1)
        pref = lax.convert_element_type(hb1 * 4096 + hb2, jnp.uint32)

        @pl.when(wid == 0)
        def _():
            z = jnp.zeros((L,), jnp.int32)
            sv = jnp.where(li == 0, hb1,
                           jnp.where(li == 1, ab1,
                                     jnp.where(li == 2, hb2,
                                               jnp.where(li == 3, ab2, z))))
            selbuf[pl.ds(0, L)] = sv
            pltpu.sync_copy(selbuf, sel2_hbm)

        pltpu.sync_copy(cnt_hbm.at[pl.ds(wid * L, L)], selbuf)
        n_w = _lane0(selbuf[pl.ds(0, L)])
        pltpu.sync_copy(lv_hbm.at[pl.ds(wid * _CAP, _CAP)], lbuf)
        _zero(subhist, L * 256)
        ones = jnp.ones((L,), jnp.int32)
        for jv in range(_CAPV):
            kui = lbuf[pl.ds(jv * L, L)]
            ku = lax.bitcast_convert_type(kui, jnp.uint32)
            valid = (jnp.int32(jv * L) + li) < n_w
            selm = valid & ((ku >> jnp.uint32(8)) == pref)
            bkt = lax.convert_element_type(ku & jnp.uint32(0xFF), jnp.int32)
            plsc.addupdate_scatter(subhist, [li * jnp.int32(256) + bkt],
                                   ones, mask=selm)
        _lane_merge(subhist, mg, 256)
        pltpu.sync_copy(mg.at[pl.ds(0, 256)],
                        h3_hbm.at[pl.ds(wid * 256, 256)])

    @functools.partial(
        pl.kernel,
        out_type=jax.ShapeDtypeStruct((_B, _H), jnp.float32),
        mesh=mesh,
        compiler_params=cp,
        scratch_types=[
            pltpu.VMEM((_CHUNK,), jnp.float32),
            pltpu.VMEM((_CHUNK,), jnp.float32),
            pltpu.VMEM((16384,), jnp.int32),
            pltpu.VMEM((256,), jnp.int32),
            pltpu.VMEM((256,), jnp.int32),
            pltpu.VMEM((L,), jnp.int32),
            pltpu.VMEM((_CAP,), jnp.int32),
            pltpu.VMEM((_CAP,), jnp.int32),
            pltpu.VMEM((_CAP,), jnp.float32),
            pltpu.VMEM((_CAP,), jnp.int32),
            pltpu.SemaphoreType.DMA,
            pltpu.SemaphoreType.DMA,
        ],
    )
    def pass4n(lv_hbm, lix_hbm, cnt_hbm, h3_hbm, sel2_hbm, y_hbm,
               bout0, bout1, tmp, m3, gsum, selbuf, lbuf, lxbuf, fbuf, kbuf,
               ws0, ws1):
        wid = _wid()
        li = _lanes()

        pltpu.sync_copy(sel2_hbm, selbuf)
        sv2 = selbuf[pl.ds(0, L)]
        hb1 = _at(sv2, 0)
        ab1 = _at(sv2, 1)
        hb2 = _at(sv2, 2)
        ab2 = _at(sv2, 3)

        pltpu.sync_copy(h3_hbm, tmp.at[pl.ds(0, NW * 256)])

        def b3(g, _):
            acc = jnp.zeros((L,), jnp.int32)
            for w in range(NW):
                acc = acc + tmp[pl.ds(w * 256 + g * L, L)]
            m3[pl.ds(g * L, L)] = acc
            return 0

        lax.fori_loop(0, 256 // L, b3, 0)
        kkt3 = jnp.int32(_KK) - ab1 - ab2
        hb3, ab3 = _desc_select(m3, gsum, 256, kkt3)

        t = ((lax.convert_element_type(hb1, jnp.uint32) << jnp.uint32(20))
             | (lax.convert_element_type(hb2, jnp.uint32) << jnp.uint32(8))
             | lax.convert_element_type(hb3, jnp.uint32))
        n_keep_ties = kkt3 - ab3

        cw_lo = plsc.load_gather(tmp, [li * jnp.int32(256) + hb3])
        cw_hi = plsc.load_gather(
            tmp, [(li + jnp.int32(16)) * jnp.int32(256) + hb3])
        cs_lo = plsc.cumsum(cw_lo)
        cs_hi = plsc.cumsum(cw_hi) + _scal(cs_lo)
        my_cw = jnp.where(wid < 16, _at(cw_lo, wid), _at(cw_hi, wid - 16))
        my_incl = jnp.where(wid < 16, _at(cs_lo, wid), _at(cs_hi, wid - 16))
        before_w = my_incl - my_cw
        budget = jnp.clip(n_keep_ties - before_w, 0, my_cw)

        pltpu.sync_copy(cnt_hbm.at[pl.ds(wid * L, L)], selbuf)
        n_w = _lane0(selbuf[pl.ds(0, L)])
        pltpu.sync_copy(lv_hbm.at[pl.ds(wid * _CAP, _CAP)], lbuf)
        pltpu.sync_copy(lix_hbm.at[pl.ds(wid * _CAP, _CAP)], lxbuf)

        one_i = jnp.ones((L,), jnp.int32)
        zero_i = jnp.zeros((L,), jnp.int32)
        r = jnp.int32(0)
        for jv in range(_CAPV):
            kui = lbuf[pl.ds(jv * L, L)]
            ku = lax.bitcast_convert_type(kui, jnp.uint32)
            valid = (jnp.int32(jv * L) + li) < n_w
            tie = valid & (ku == t)
            cs = plsc.cumsum(jnp.where(tie, one_i, zero_i))
            keep = (valid & (ku > t)) | (tie & ((r + cs) <= budget))
            kbuf[pl.ds(jv * L, L)] = jnp.where(keep, one_i, zero_i)
            fbuf[pl.ds(jv * L, L)] = _inv_ku16(ku)
            r = r + _scal(cs)

        zf = jnp.zeros((L,), jnp.float32)
        bouts = [bout0, bout1]
        wsems = [ws0, ws1]
        wh = [None, None]
        for c in range(_NCHUNKS):
            b = c % 2
            if wh[b] is not None:
                wh[b].wait()

            def zb(i, _):
                bouts[b][pl.ds(i * L, L)] = zf
                return 0

            lax.fori_loop(0, _CHUNK // L, zb, 0, unroll=8)
            for jv in range(_CAPV):
                lx = lxbuf[pl.ds(jv * L, L)]
                keep = kbuf[pl.ds(jv * L, L)] > 0
                m = keep & ((lx >> 14) == c)
                off = lx & jnp.int32(16383)
                plsc.store_scatter(bouts[b], [off], fbuf[pl.ds(jv * L, L)],
                                   mask=m)
            r_, c_ = _chunk_rc(wid, c)
            wh[b] = pltpu.async_copy(
                bouts[b], y_hbm.at[r_, pl.ds(c_, _CHUNK)], wsems[b])
        for b in range(2):
            if wh[b] is not None:
                wh[b].wait()

    return pass1, pass2, pass3, pass4, pass3n, pass4n


@jax.jit
def kernel(hidden_preactivation_BH):
    pass1, pass2, pass3, pass4, pass3n, pass4n = _build_passes()
    x = hidden_preactivation_BH
    h1 = pass1(x)
    h2, sel1, lv, lix, cnt = pass2(x, h1)
    overflow = jnp.any(cnt.reshape(NW, L)[:, 0] > _CAP)

    def fast(op):
        x_, h2_, sel1_, lv_, lix_, cnt_ = op
        h3, sel2 = pass3n(h2_, sel1_, lv_, cnt_)
        return pass4n(lv_, lix_, cnt_, h3, sel2)

    def full(op):
        x_, h2_, sel1_, lv_, lix_, cnt_ = op
        h3, sel2 = pass3(x_, h2_, sel1_)
        return pass4(x_, h3, sel2)

    return lax.cond(overflow, full, fast, (x, h2, sel1, lv, lix, cnt))
